# baseline (speedup 1.0000x reference)
import functools
import jax
import jax.numpy as jnp
from jax.experimental import pallas as pl
from jax.experimental.pallas import tpu as pltpu

N_NODES_BLK = 2000


def _proj_body(x_ref, w0_ref, w1_ref, a2_ref, h0_ref, h1_ref, st_ref):
    x = x_ref[...]
    h0 = jnp.dot(x, w0_ref[...], preferred_element_type=jnp.float32)
    h1 = jnp.dot(x, w1_ref[...], preferred_element_type=jnp.float32)
    h0_ref[...] = h0
    h1_ref[...] = h1
    a2 = a2_ref[...]
    st0 = jnp.dot(h0, a2[:, 0:2], preferred_element_type=jnp.float32)
    st1 = jnp.dot(h1, a2[:, 2:4], preferred_element_type=jnp.float32)
    st_ref[...] = jnp.concatenate(
        [st0, st1, jnp.zeros_like(st0), jnp.zeros_like(st0)], axis=-1)


def _project(x, W0, W1, a0, a1):
    N, D = x.shape
    a2 = jnp.stack([a0[:D], a0[D:2 * D], a1[:D], a1[D:2 * D]], axis=-1)
    a2 = jnp.pad(a2, ((0, 0), (0, 4)))
    grid = (N // N_NODES_BLK,)
    h0, h1, st = pl.pallas_call(
        _proj_body,
        grid=grid,
        in_specs=[
            pl.BlockSpec((N_NODES_BLK, D), lambda i: (i, 0)),
            pl.BlockSpec((D, D), lambda i: (0, 0)),
            pl.BlockSpec((D, D), lambda i: (0, 0)),
            pl.BlockSpec((D, 8), lambda i: (0, 0)),
        ],
        out_specs=[
            pl.BlockSpec((N_NODES_BLK, D), lambda i: (i, 0)),
            pl.BlockSpec((N_NODES_BLK, D), lambda i: (i, 0)),
            pl.BlockSpec((N_NODES_BLK, 8), lambda i: (i, 0)),
        ],
        out_shape=[
            jax.ShapeDtypeStruct((N, D), jnp.float32),
            jax.ShapeDtypeStruct((N, D), jnp.float32),
            jax.ShapeDtypeStruct((N, 8), jnp.float32),
        ],
    )(x, W0, W1, a2)
    return h0, h1, st


def _attend_sparse(h, s, t, edge_index, edge_attr, a_last, negative_slope=0.2):
    N = h.shape[0]
    src = edge_index[0]
    dst = edge_index[1]
    ea1 = edge_attr[:, 0]
    e = s[src] + t[dst] + ea1 * a_last
    e = jnp.where(e >= 0, e, negative_slope * e)
    e_max = jax.ops.segment_max(e, dst, num_segments=N)
    e_max = jnp.where(jnp.isfinite(e_max), e_max, 0.0)
    alpha_un = jnp.exp(e - e_max[dst])
    alpha_sum = jax.ops.segment_sum(alpha_un, dst, num_segments=N)
    alpha_sum = jnp.clip(alpha_sum, 1e-12, None)
    alpha = alpha_un / alpha_sum[dst]
    ea_weight = jnp.clip(jnp.abs(ea1), 0.01, None)
    msg = (alpha * ea_weight)[:, None] * h[src]
    out = jax.ops.segment_sum(msg, dst, num_segments=N)
    return out, alpha


def kernel(x, edge_index_r0, edge_attr_r0, edge_index_r1, edge_attr_r1,
           W0, W1, a0, a1, relation_logits, bias):
    D = x.shape[1]
    h0, h1, st = _project(x, W0, W1, a0, a1)
    o0, alpha0 = _attend_sparse(h0, st[:, 0], st[:, 1], edge_index_r0,
                                edge_attr_r0, a0[2 * D])
    o1, alpha1 = _attend_sparse(h1, st[:, 2], st[:, 3], edge_index_r1,
                                edge_attr_r1, a1[2 * D])
    weights = jax.nn.softmax(relation_logits, axis=0)
    out = weights[0] * o0 + weights[1] * o1 + bias
    return (out, alpha0, alpha1)


# speedup vs baseline: 9.2214x; 9.2214x over previous
import functools
import jax
import jax.numpy as jnp
from jax import lax
from jax.experimental import pallas as pl
from jax.experimental.pallas import tpu as pltpu
from jax.experimental.pallas import tpu_sc as plsc

N = 50000
E = 400000
D = 128
NPAD = 50176
HNP = NPAD // 2
EPAD = 401408
NSUB = 16
NC = 2
NODE_TILE = NPAD // NSUB
EA_TILE = EPAD // NSUB
CHUNK_A = 1792
NCHUNK_A = EA_TILE // CHUNK_A
GSUB = CHUNK_A // 128
EW_TILE = EPAD // (NSUB * NC)
NCHUNK_W = EW_TILE // CHUNK_A

N_NODES_BLK = 2000


def _proj_body(x_ref, w0_ref, w1_ref, a2_ref, h0_ref, h1_ref, st_ref):
    x = x_ref[...]
    h0 = jnp.dot(x, w0_ref[...], preferred_element_type=jnp.float32)
    h1 = jnp.dot(x, w1_ref[...], preferred_element_type=jnp.float32)
    h0_ref[...] = h0
    h1_ref[...] = h1
    a2 = a2_ref[...]
    st0 = jnp.dot(h0, a2[:, 0:2], preferred_element_type=jnp.float32)
    st1 = jnp.dot(h1, a2[:, 2:4], preferred_element_type=jnp.float32)
    st_ref[...] = jnp.concatenate(
        [st0, st1, jnp.zeros_like(st0), jnp.zeros_like(st0)], axis=-1)


def _project(x, W0, W1, a0, a1):
    a2 = jnp.stack([a0[:D], a0[D:2 * D], a1[:D], a1[D:2 * D]], axis=-1)
    a2 = jnp.pad(a2, ((0, 0), (0, 4)))
    grid = (N // N_NODES_BLK,)
    h0, h1, st = pl.pallas_call(
        _proj_body,
        grid=grid,
        in_specs=[
            pl.BlockSpec((N_NODES_BLK, D), lambda i: (i, 0)),
            pl.BlockSpec((D, D), lambda i: (0, 0)),
            pl.BlockSpec((D, D), lambda i: (0, 0)),
            pl.BlockSpec((D, 8), lambda i: (0, 0)),
        ],
        out_specs=[
            pl.BlockSpec((N_NODES_BLK, D), lambda i: (i, 0)),
            pl.BlockSpec((N_NODES_BLK, D), lambda i: (i, 0)),
            pl.BlockSpec((N_NODES_BLK, 8), lambda i: (i, 0)),
        ],
        out_shape=[
            jax.ShapeDtypeStruct((N, D), jnp.float32),
            jax.ShapeDtypeStruct((N, D), jnp.float32),
            jax.ShapeDtypeStruct((N, 8), jnp.float32),
        ],
    )(x, W0, W1, a2)
    return h0, h1, st


def _softmax_body(src_hbm, dst_hbm, ea_hbm, s_hbm, t_hbm, par_hbm,
                  alpha_hbm, coef_hbm,
                  idx_s, idx_d, ea_v, g1_v, g2_v, g3_v, g4_v,
                  acc_v, comb_v, tmp_v, pv_v,
                  s_sp, t_sp, emax_sp, inv_sp, red_sp, sem, sem2):
    cid = lax.axis_index("c")
    sid = lax.axis_index("s")
    lanes = lax.iota(jnp.int32, 16)
    one = jnp.ones((16,), jnp.int32)
    neg_inf = jnp.full((16,), -jnp.inf, jnp.float32)
    zero16 = jnp.zeros((16,), jnp.float32)

    obase = pl.multiple_of(sid * NODE_TILE, 8)
    pltpu.sync_copy(s_hbm.at[pl.ds(obase, NODE_TILE)], comb_v)
    pltpu.sync_copy(comb_v, s_sp.at[pl.ds(obase, NODE_TILE)])
    pltpu.sync_copy(t_hbm.at[pl.ds(obase, NODE_TILE)], comb_v)
    pltpu.sync_copy(comb_v, t_sp.at[pl.ds(obase, NODE_TILE)])
    pltpu.sync_copy(par_hbm, pv_v)

    def initm(i, _):
        acc_v[pl.ds(i * 16, 16)] = neg_inf
        return 0
    lax.fori_loop(0, NPAD // 16, initm, 0)
    plsc.subcore_barrier()

    a256 = pv_v[...]
    ebase = pl.multiple_of(sid * EA_TILE, 8)

    def seg_update(idx, val, bits0, is_max):
        def cond(b):
            return b > 0

        def step(b):
            act = ((b >> lanes) & 1) != 0
            cur = plsc.load_gather(acc_v, [idx], mask=act)
            plsc.store_scatter(acc_v, [idx], plsc.bitcast(lanes, jnp.float32),
                               mask=act)
            back = plsc.bitcast(plsc.load_gather(acc_v, [idx], mask=act),
                                jnp.int32)
            win = (back == lanes) & act
            newv = jnp.maximum(cur, val) if is_max else cur + val
            plsc.store_scatter(acc_v, [idx], newv, mask=win)
            rem = act & jnp.logical_not(win)
            return jnp.sum(jnp.where(rem, one << lanes, 0))

        lax.while_loop(cond, step, bits0)

    def gather128(table_sp, idx_ref, out_ref, s):
        descs = []
        for g in range(GSUB):
            descs.append(pltpu.async_copy(
                table_sp.at[idx_ref.at[pl.ds(g * 128, 128)]],
                out_ref.at[pl.ds(g * 128, 128)], s))
        return descs

    def load_edges(off):
        pltpu.sync_copy(src_hbm.at[pl.ds(off, CHUNK_A)], idx_s)
        pltpu.sync_copy(dst_hbm.at[pl.ds(off, CHUNK_A)], idx_d)
        pltpu.sync_copy(ea_hbm.at[pl.ds(off, CHUNK_A)], ea_v)
        ds1 = gather128(s_sp, idx_s, g1_v, sem)
        ds2 = gather128(t_sp, idx_d, g2_v, sem2)
        for d in ds1 + ds2:
            d.wait()

    def compute_e(j):
        sv = g1_v[pl.ds(j * 16, 16)]
        tv = g2_v[pl.ds(j * 16, 16)]
        eav = ea_v[pl.ds(j * 16, 16)]
        e = sv + tv + eav * a256
        return jnp.where(e >= 0, e, e * jnp.float32(0.2))

    def combine(is_max, dst_sp):
        for half in range(2):
            hbase = half * HNP
            pltpu.sync_copy(
                acc_v.at[pl.ds(hbase, HNP)],
                red_sp.at[pl.ds(pl.multiple_of(sid * HNP, 8), HNP)])
            plsc.subcore_barrier()

            @pl.when((sid // 8) == half)
            def _():
                lbase = pl.multiple_of((sid - half * 8) * NODE_TILE, 8)

                def cinit(i, _):
                    comb_v[pl.ds(i * 16, 16)] = (neg_inf if is_max
                                                 else zero16)
                    return 0
                lax.fori_loop(0, NODE_TILE // 16, cinit, 0)

                def creduce(t, _):
                    pltpu.sync_copy(
                        red_sp.at[pl.ds(
                            pl.multiple_of(t * HNP, 8) + lbase, NODE_TILE)],
                        tmp_v)

                    def vred(i, _):
                        a = comb_v[pl.ds(i * 16, 16)]
                        b = tmp_v[pl.ds(i * 16, 16)]
                        comb_v[pl.ds(i * 16, 16)] = (
                            jnp.maximum(a, b) if is_max else a + b)
                        return 0
                    lax.fori_loop(0, NODE_TILE // 16, vred, 0)
                    return 0
                lax.fori_loop(0, NSUB, creduce, 0)

                def cfin(i, _):
                    v = comb_v[pl.ds(i * 16, 16)]
                    if is_max:
                        v = jnp.where(v == neg_inf, zero16, v)
                    else:
                        v = jnp.float32(1.0) / jnp.maximum(
                            v, jnp.full((16,), 1e-12, jnp.float32))
                    comb_v[pl.ds(i * 16, 16)] = v
                    return 0
                lax.fori_loop(0, NODE_TILE // 16, cfin, 0)
                pltpu.sync_copy(comb_v, dst_sp.at[pl.ds(obase, NODE_TILE)])
            plsc.subcore_barrier()

    def chunk1(k, _):
        off = pl.multiple_of(ebase + k * CHUNK_A, 8)
        load_edges(off)

        def vloop(j, _):
            e = compute_e(j)
            idx = idx_d[pl.ds(j * 16, 16)]
            valid = (off + j * 16 + lanes) < E
            bits0 = jnp.sum(jnp.where(valid, one << lanes, 0))
            seg_update(idx, e, bits0, True)
            return 0

        lax.fori_loop(0, CHUNK_A // 16, vloop, 0)
        return 0

    lax.fori_loop(0, NCHUNK_A, chunk1, 0)

    combine(True, emax_sp)

    def initz(i, _):
        acc_v[pl.ds(i * 16, 16)] = zero16
        return 0
    lax.fori_loop(0, NPAD // 16, initz, 0)
    plsc.subcore_barrier()

    def chunk2(k, _):
        off = pl.multiple_of(ebase + k * CHUNK_A, 8)
        load_edges(off)
        for d in gather128(emax_sp, idx_d, g3_v, sem):
            d.wait()

        def vloop(j, _):
            e = compute_e(j)
            em = g3_v[pl.ds(j * 16, 16)]
            au = jnp.exp(e - em)
            idx = idx_d[pl.ds(j * 16, 16)]
            valid = (off + j * 16 + lanes) < E
            bits0 = jnp.sum(jnp.where(valid, one << lanes, 0))
            seg_update(idx, au, bits0, False)
            return 0

        lax.fori_loop(0, CHUNK_A // 16, vloop, 0)
        return 0

    lax.fori_loop(0, NCHUNK_A, chunk2, 0)

    combine(False, inv_sp)

    wbase = sid * EA_TILE + cid * EW_TILE

    def chunk3(k, _):
        off = pl.multiple_of(wbase + k * CHUNK_A, 8)
        load_edges(off)
        ds3 = gather128(emax_sp, idx_d, g3_v, sem)
        ds4 = gather128(inv_sp, idx_d, g4_v, sem2)
        for d in ds3 + ds4:
            d.wait()

        def vloop(j, _):
            e = compute_e(j)
            em = g3_v[pl.ds(j * 16, 16)]
            iv = g4_v[pl.ds(j * 16, 16)]
            eav = ea_v[pl.ds(j * 16, 16)]
            valid = (off + j * 16 + lanes) < E
            alpha = jnp.where(valid, jnp.exp(e - em) * iv, zero16)
            ew = jnp.maximum(jnp.abs(eav), jnp.full((16,), 0.01, jnp.float32))
            g1_v[pl.ds(j * 16, 16)] = alpha
            g2_v[pl.ds(j * 16, 16)] = alpha * ew
            return 0

        lax.fori_loop(0, CHUNK_A // 16, vloop, 0)
        pltpu.sync_copy(g1_v, alpha_hbm.at[pl.ds(off, CHUNK_A)])
        pltpu.sync_copy(g2_v, coef_hbm.at[pl.ds(off, CHUNK_A)])
        return 0

    lax.fori_loop(0, NCHUNK_W, chunk3, 0)


def _sc_softmax(src, dst, ea, s, t, par):
    mesh = plsc.VectorSubcoreMesh(core_axis_name="c", subcore_axis_name="s")
    kern = functools.partial(
        pl.kernel,
        out_type=[
            jax.ShapeDtypeStruct((EPAD,), jnp.float32),
            jax.ShapeDtypeStruct((EPAD,), jnp.float32),
        ],
        mesh=mesh,
        compiler_params=pltpu.CompilerParams(needs_layout_passes=False),
        scratch_types=[
            pltpu.VMEM((CHUNK_A,), jnp.int32),
            pltpu.VMEM((CHUNK_A,), jnp.int32),
            pltpu.VMEM((CHUNK_A,), jnp.float32),
            pltpu.VMEM((CHUNK_A,), jnp.float32),
            pltpu.VMEM((CHUNK_A,), jnp.float32),
            pltpu.VMEM((CHUNK_A,), jnp.float32),
            pltpu.VMEM((CHUNK_A,), jnp.float32),
            pltpu.VMEM((NPAD,), jnp.float32),
            pltpu.VMEM((NODE_TILE,), jnp.float32),
            pltpu.VMEM((NODE_TILE,), jnp.float32),
            pltpu.VMEM((16,), jnp.float32),
            pltpu.VMEM_SHARED((NPAD,), jnp.float32),
            pltpu.VMEM_SHARED((NPAD,), jnp.float32),
            pltpu.VMEM_SHARED((NPAD,), jnp.float32),
            pltpu.VMEM_SHARED((NPAD,), jnp.float32),
            pltpu.VMEM_SHARED((NSUB * HNP,), jnp.float32),
            pltpu.SemaphoreType.DMA,
            pltpu.SemaphoreType.DMA,
        ],
    )(_softmax_body)
    return kern(src, dst, ea, s, t, par)


NPASS = 8
PASS_ROWS = NPAD // NPASS
EB_TILE = EPAD // (NSUB * NC)
CHUNK_B = 1568
NCHUNK_B = EB_TILE // CHUNK_B
STAGE_B = 1664
DRAIN_W = 56
TILE_ROWS = PASS_ROWS // NSUB


def _agg_body(src_hbm, dst_hbm, coef_hbm, h_hbm, opart_hbm,
              c_src, c_dst, c_cof, st_src, st_dst, st_cof,
              blki_v, blkd_v, blkc_v, rows_v, zero_v, acc_sp, sem):
    cid = lax.axis_index("c")
    sid = lax.axis_index("s")
    lanes = lax.iota(jnp.int32, 16)
    zero16 = jnp.zeros((16,), jnp.float32)
    wid = cid * NSUB + sid
    tbase = pl.multiple_of(wid * EB_TILE, 8)
    rb0 = sid * TILE_ROWS

    def zinit(r, _):
        for j in range(8):
            zero_v[r, pl.ds(j * 16, 16)] = zero16
        return 0
    lax.fori_loop(0, DRAIN_W, zinit, 0)

    for p in range(NPASS):
        prow_base = p * PASS_ROWS
        for w in range(7):
            rs = pl.multiple_of(rb0 + w * DRAIN_W, 8)
            pltpu.sync_copy(zero_v, acc_sp.at[pl.ds(rs, DRAIN_W)])
        plsc.subcore_barrier()

        def chunkb(k, _):
            off = pl.multiple_of(tbase + k * CHUNK_B, 8)
            pltpu.sync_copy(src_hbm.at[pl.ds(off, CHUNK_B)], c_src)
            pltpu.sync_copy(dst_hbm.at[pl.ds(off, CHUNK_B)], c_dst)
            pltpu.sync_copy(coef_hbm.at[pl.ds(off, CHUNK_B)], c_cof)

            def vstage(j, cnt):
                dl = c_dst[pl.ds(j * 16, 16)] - prow_base
                m = (dl >= 0) & (dl < PASS_ROWS)
                plsc.store_compressed(st_src.at[pl.ds(cnt, 16)],
                                      c_src[pl.ds(j * 16, 16)], mask=m)
                plsc.store_compressed(st_dst.at[pl.ds(cnt, 16)], dl, mask=m)
                plsc.store_compressed(st_cof.at[pl.ds(cnt, 16)],
                                      c_cof[pl.ds(j * 16, 16)], mask=m)
                return cnt + jnp.sum(m.astype(jnp.int32))

            cnt = lax.fori_loop(0, CHUNK_B // 16, vstage, 0)
            nblk = (cnt + 127) // 128

            def gblk(b, _):
                boff = b * 128
                for j in range(8):
                    pos = boff + j * 16 + lanes
                    vv = pos < cnt
                    sidx = st_src[pl.ds(boff + j * 16, 16)]
                    didx = st_dst[pl.ds(boff + j * 16, 16)]
                    cv = st_cof[pl.ds(boff + j * 16, 16)]
                    fb = j * 16 + lanes
                    blki_v[pl.ds(j * 16, 16)] = jnp.where(vv, sidx, fb)
                    blkd_v[pl.ds(j * 16, 16)] = jnp.where(vv, didx, fb)
                    blkc_v[pl.ds(j * 16, 16)] = jnp.where(vv, cv, zero16)
                pltpu.async_copy(h_hbm.at[blki_v], rows_v, sem).wait()

                def scale(r, _):
                    cvec = plsc.load_gather(
                        blkc_v, [jnp.full((16,), r, jnp.int32)])
                    for jj in range(8):
                        rows_v[r, pl.ds(jj * 16, 16)] = (
                            rows_v[r, pl.ds(jj * 16, 16)] * cvec)
                    return 0
                lax.fori_loop(0, 128, scale, 0)
                pltpu.sync_copy(rows_v, acc_sp.at[blkd_v], add=True)
                return 0

            lax.fori_loop(0, nblk, gblk, 0)
            return 0

        lax.fori_loop(0, NCHUNK_B, chunkb, 0)
        plsc.subcore_barrier()

        for w in range(7):
            rs = pl.multiple_of(rb0 + w * DRAIN_W, 8)
            pltpu.sync_copy(acc_sp.at[pl.ds(rs, DRAIN_W)],
                            rows_v.at[pl.ds(0, DRAIN_W)])
            pltpu.sync_copy(
                rows_v.at[pl.ds(0, DRAIN_W)],
                opart_hbm.at[cid, pl.ds(pl.multiple_of(prow_base, 8) + rs,
                                        DRAIN_W)])
        plsc.subcore_barrier()


def _sc_aggregate(src, dst, coef, h):
    mesh = plsc.VectorSubcoreMesh(core_axis_name="c", subcore_axis_name="s")
    kern = functools.partial(
        pl.kernel,
        out_type=[jax.ShapeDtypeStruct((NC, NPAD, D), jnp.float32)],
        mesh=mesh,
        compiler_params=pltpu.CompilerParams(needs_layout_passes=False),
        scratch_types=[
            pltpu.VMEM((CHUNK_B,), jnp.int32),
            pltpu.VMEM((CHUNK_B,), jnp.int32),
            pltpu.VMEM((CHUNK_B,), jnp.float32),
            pltpu.VMEM((STAGE_B,), jnp.int32),
            pltpu.VMEM((STAGE_B,), jnp.int32),
            pltpu.VMEM((STAGE_B,), jnp.float32),
            pltpu.VMEM((128,), jnp.int32),
            pltpu.VMEM((128,), jnp.int32),
            pltpu.VMEM((128,), jnp.float32),
            pltpu.VMEM((128, D), jnp.float32),
            pltpu.VMEM((DRAIN_W, D), jnp.float32),
            pltpu.VMEM_SHARED((PASS_ROWS, D), jnp.float32),
            pltpu.SemaphoreType.DMA,
        ],
    )(_agg_body)
    (opart,) = kern(src, dst, coef, h)
    return opart


def _combine_body(w_ref, o0_ref, o1_ref, b_ref, out_ref):
    o0 = o0_ref[0] + o0_ref[1]
    o1 = o1_ref[0] + o1_ref[1]
    out_ref[...] = w_ref[0] * o0 + w_ref[1] * o1 + b_ref[...]


def _combine(w, opart0, opart1, bias):
    grid = (N // N_NODES_BLK,)
    return pl.pallas_call(
        _combine_body,
        grid=grid,
        in_specs=[
            pl.BlockSpec(memory_space=pltpu.SMEM),
            pl.BlockSpec((NC, N_NODES_BLK, D), lambda i: (0, i, 0)),
            pl.BlockSpec((NC, N_NODES_BLK, D), lambda i: (0, i, 0)),
            pl.BlockSpec((1, D), lambda i: (0, 0)),
        ],
        out_specs=pl.BlockSpec((N_NODES_BLK, D), lambda i: (i, 0)),
        out_shape=jax.ShapeDtypeStruct((N, D), jnp.float32),
    )(w, opart0, opart1, bias.reshape(1, D))


def _attend_rel(h, s, t, edge_index, edge_attr, a_last):
    src = jnp.pad(edge_index[0], (0, EPAD - E))
    dst = jnp.pad(edge_index[1], (0, EPAD - E))
    ea1 = jnp.pad(edge_attr[:, 0], (0, EPAD - E))
    sp = jnp.pad(s, (0, NPAD - N))
    tp = jnp.pad(t, (0, NPAD - N))
    par = jnp.full((16,), a_last, jnp.float32)
    alpha_p, coef_p = _sc_softmax(src, dst, ea1, sp, tp, par)
    opart = _sc_aggregate(src, dst, coef_p, h)
    return opart, alpha_p[:E]


def kernel(x, edge_index_r0, edge_attr_r0, edge_index_r1, edge_attr_r1,
           W0, W1, a0, a1, relation_logits, bias):
    h0, h1, st = _project(x, W0, W1, a0, a1)
    op0, alpha0 = _attend_rel(h0, st[:, 0], st[:, 1], edge_index_r0,
                              edge_attr_r0, a0[2 * D])
    op1, alpha1 = _attend_rel(h1, st[:, 2], st[:, 3], edge_index_r1,
                              edge_attr_r1, a1[2 * D])
    weights = jax.nn.softmax(relation_logits, axis=0)
    out = _combine(weights, op0, op1, bias)
    return (out, alpha0, alpha1)


# speedup vs baseline: 9.7899x; 1.0616x over previous
import functools
import jax
import jax.numpy as jnp
from jax import lax
from jax.experimental import pallas as pl
from jax.experimental.pallas import tpu as pltpu
from jax.experimental.pallas import tpu_sc as plsc

N = 50000
E = 400000
D = 128
NPAD = 50176
HNP = NPAD // 2
EPAD = 401408
NSUB = 16
NC = 2
NODE_TILE = NPAD // NSUB
EA_TILE = EPAD // NSUB
CHUNK_A = 1792
NCHUNK_A = EA_TILE // CHUNK_A
GSUB = CHUNK_A // 128
EW_TILE = EPAD // (NSUB * NC)
NCHUNK_W = EW_TILE // CHUNK_A

N_NODES_BLK = 2000


def _proj_body(x_ref, w0_ref, w1_ref, a2_ref, h0_ref, h1_ref, st_ref):
    x = x_ref[...]
    h0 = jnp.dot(x, w0_ref[...], preferred_element_type=jnp.float32)
    h1 = jnp.dot(x, w1_ref[...], preferred_element_type=jnp.float32)
    h0_ref[...] = h0
    h1_ref[...] = h1
    a2 = a2_ref[...]
    st0 = jnp.dot(h0, a2[:, 0:2], preferred_element_type=jnp.float32)
    st1 = jnp.dot(h1, a2[:, 2:4], preferred_element_type=jnp.float32)
    st_ref[...] = jnp.concatenate(
        [st0, st1, jnp.zeros_like(st0), jnp.zeros_like(st0)], axis=-1)


def _project(x, W0, W1, a0, a1):
    a2 = jnp.stack([a0[:D], a0[D:2 * D], a1[:D], a1[D:2 * D]], axis=-1)
    a2 = jnp.pad(a2, ((0, 0), (0, 4)))
    grid = (N // N_NODES_BLK,)
    h0, h1, st = pl.pallas_call(
        _proj_body,
        grid=grid,
        in_specs=[
            pl.BlockSpec((N_NODES_BLK, D), lambda i: (i, 0)),
            pl.BlockSpec((D, D), lambda i: (0, 0)),
            pl.BlockSpec((D, D), lambda i: (0, 0)),
            pl.BlockSpec((D, 8), lambda i: (0, 0)),
        ],
        out_specs=[
            pl.BlockSpec((N_NODES_BLK, D), lambda i: (i, 0)),
            pl.BlockSpec((N_NODES_BLK, D), lambda i: (i, 0)),
            pl.BlockSpec((N_NODES_BLK, 8), lambda i: (i, 0)),
        ],
        out_shape=[
            jax.ShapeDtypeStruct((N, D), jnp.float32),
            jax.ShapeDtypeStruct((N, D), jnp.float32),
            jax.ShapeDtypeStruct((N, 8), jnp.float32),
        ],
    )(x, W0, W1, a2)
    return h0, h1, st


def _softmax_body(src_hbm, dst_hbm, ea_hbm, s_hbm, t_hbm, par_hbm,
                  alpha_hbm, coef_hbm,
                  idx_s, idx_d, ea_v, g1_v, g2_v, g3_v, g4_v,
                  acc_v, comb_v, tmp_v, pv_v,
                  s_sp, t_sp, emax_sp, inv_sp, red_sp, sem, sem2):
    cid = lax.axis_index("c")
    sid = lax.axis_index("s")
    lanes = lax.iota(jnp.int32, 16)
    one = jnp.ones((16,), jnp.int32)
    neg_inf = jnp.full((16,), -jnp.inf, jnp.float32)
    zero16 = jnp.zeros((16,), jnp.float32)

    obase = pl.multiple_of(sid * NODE_TILE, 8)
    pltpu.sync_copy(s_hbm.at[pl.ds(obase, NODE_TILE)], comb_v)
    pltpu.sync_copy(comb_v, s_sp.at[pl.ds(obase, NODE_TILE)])
    pltpu.sync_copy(t_hbm.at[pl.ds(obase, NODE_TILE)], comb_v)
    pltpu.sync_copy(comb_v, t_sp.at[pl.ds(obase, NODE_TILE)])
    pltpu.sync_copy(par_hbm, pv_v)

    def initm(i, _):
        acc_v[pl.ds(i * 16, 16)] = neg_inf
        return 0
    lax.fori_loop(0, NPAD // 16, initm, 0)
    plsc.subcore_barrier()

    a256 = pv_v[...]
    ebase = pl.multiple_of(sid * EA_TILE, 8)

    def seg_update(idx, val, bits0, is_max):
        def cond(b):
            return b > 0

        def step(b):
            act = ((b >> lanes) & 1) != 0
            cur = plsc.load_gather(acc_v, [idx], mask=act)
            plsc.store_scatter(acc_v, [idx], plsc.bitcast(lanes, jnp.float32),
                               mask=act)
            back = plsc.bitcast(plsc.load_gather(acc_v, [idx], mask=act),
                                jnp.int32)
            win = (back == lanes) & act
            newv = jnp.maximum(cur, val) if is_max else cur + val
            plsc.store_scatter(acc_v, [idx], newv, mask=win)
            rem = act & jnp.logical_not(win)
            return jnp.sum(jnp.where(rem, one << lanes, 0))

        lax.while_loop(cond, step, bits0)

    def gather128(table_sp, idx_ref, out_ref, s):
        descs = []
        for g in range(GSUB):
            descs.append(pltpu.async_copy(
                table_sp.at[idx_ref.at[pl.ds(g * 128, 128)]],
                out_ref.at[pl.ds(g * 128, 128)], s))
        return descs

    def load_edges(off):
        pltpu.sync_copy(src_hbm.at[pl.ds(off, CHUNK_A)], idx_s)
        pltpu.sync_copy(dst_hbm.at[pl.ds(off, CHUNK_A)], idx_d)
        pltpu.sync_copy(ea_hbm.at[pl.ds(off, CHUNK_A)], ea_v)
        ds1 = gather128(s_sp, idx_s, g1_v, sem)
        ds2 = gather128(t_sp, idx_d, g2_v, sem2)
        for d in ds1 + ds2:
            d.wait()

    def compute_e(j):
        sv = g1_v[pl.ds(j * 16, 16)]
        tv = g2_v[pl.ds(j * 16, 16)]
        eav = ea_v[pl.ds(j * 16, 16)]
        e = sv + tv + eav * a256
        return jnp.where(e >= 0, e, e * jnp.float32(0.2))

    def combine(is_max, dst_sp):
        for half in range(2):
            hbase = half * HNP
            pltpu.sync_copy(
                acc_v.at[pl.ds(hbase, HNP)],
                red_sp.at[pl.ds(pl.multiple_of(sid * HNP, 8), HNP)])
            plsc.subcore_barrier()

            @pl.when((sid // 8) == half)
            def _():
                lbase = pl.multiple_of((sid - half * 8) * NODE_TILE, 8)

                def cinit(i, _):
                    comb_v[pl.ds(i * 16, 16)] = (neg_inf if is_max
                                                 else zero16)
                    return 0
                lax.fori_loop(0, NODE_TILE // 16, cinit, 0)

                def creduce(t, _):
                    pltpu.sync_copy(
                        red_sp.at[pl.ds(
                            pl.multiple_of(t * HNP, 8) + lbase, NODE_TILE)],
                        tmp_v)

                    def vred(i, _):
                        a = comb_v[pl.ds(i * 16, 16)]
                        b = tmp_v[pl.ds(i * 16, 16)]
                        comb_v[pl.ds(i * 16, 16)] = (
                            jnp.maximum(a, b) if is_max else a + b)
                        return 0
                    lax.fori_loop(0, NODE_TILE // 16, vred, 0)
                    return 0
                lax.fori_loop(0, NSUB, creduce, 0)

                def cfin(i, _):
                    v = comb_v[pl.ds(i * 16, 16)]
                    if is_max:
                        v = jnp.where(v == neg_inf, zero16, v)
                    else:
                        v = jnp.float32(1.0) / jnp.maximum(
                            v, jnp.full((16,), 1e-12, jnp.float32))
                    comb_v[pl.ds(i * 16, 16)] = v
                    return 0
                lax.fori_loop(0, NODE_TILE // 16, cfin, 0)
                pltpu.sync_copy(comb_v, dst_sp.at[pl.ds(obase, NODE_TILE)])
            plsc.subcore_barrier()

    def chunk1(k, _):
        off = pl.multiple_of(ebase + k * CHUNK_A, 8)
        load_edges(off)

        def vloop(j, _):
            e = compute_e(j)
            idx = idx_d[pl.ds(j * 16, 16)]
            valid = (off + j * 16 + lanes) < E
            bits0 = jnp.sum(jnp.where(valid, one << lanes, 0))
            seg_update(idx, e, bits0, True)
            return 0

        lax.fori_loop(0, CHUNK_A // 16, vloop, 0)
        return 0

    lax.fori_loop(0, NCHUNK_A, chunk1, 0)

    combine(True, emax_sp)

    def initz(i, _):
        acc_v[pl.ds(i * 16, 16)] = zero16
        return 0
    lax.fori_loop(0, NPAD // 16, initz, 0)
    plsc.subcore_barrier()

    def chunk2(k, _):
        off = pl.multiple_of(ebase + k * CHUNK_A, 8)
        load_edges(off)
        for d in gather128(emax_sp, idx_d, g3_v, sem):
            d.wait()

        def vloop(j, _):
            e = compute_e(j)
            em = g3_v[pl.ds(j * 16, 16)]
            au = jnp.exp(e - em)
            idx = idx_d[pl.ds(j * 16, 16)]
            valid = (off + j * 16 + lanes) < E
            bits0 = jnp.sum(jnp.where(valid, one << lanes, 0))
            seg_update(idx, au, bits0, False)
            return 0

        lax.fori_loop(0, CHUNK_A // 16, vloop, 0)
        return 0

    lax.fori_loop(0, NCHUNK_A, chunk2, 0)

    combine(False, inv_sp)

    wbase = sid * EA_TILE + cid * EW_TILE

    def chunk3(k, _):
        off = pl.multiple_of(wbase + k * CHUNK_A, 8)
        load_edges(off)
        ds3 = gather128(emax_sp, idx_d, g3_v, sem)
        ds4 = gather128(inv_sp, idx_d, g4_v, sem2)
        for d in ds3 + ds4:
            d.wait()

        def vloop(j, _):
            e = compute_e(j)
            em = g3_v[pl.ds(j * 16, 16)]
            iv = g4_v[pl.ds(j * 16, 16)]
            eav = ea_v[pl.ds(j * 16, 16)]
            valid = (off + j * 16 + lanes) < E
            alpha = jnp.where(valid, jnp.exp(e - em) * iv, zero16)
            ew = jnp.maximum(jnp.abs(eav), jnp.full((16,), 0.01, jnp.float32))
            g1_v[pl.ds(j * 16, 16)] = alpha
            g2_v[pl.ds(j * 16, 16)] = alpha * ew
            return 0

        lax.fori_loop(0, CHUNK_A // 16, vloop, 0)
        pltpu.sync_copy(g1_v, alpha_hbm.at[pl.ds(off, CHUNK_A)])
        pltpu.sync_copy(g2_v, coef_hbm.at[pl.ds(off, CHUNK_A)])
        return 0

    lax.fori_loop(0, NCHUNK_W, chunk3, 0)


def _sc_softmax(src, dst, ea, s, t, par):
    mesh = plsc.VectorSubcoreMesh(core_axis_name="c", subcore_axis_name="s")
    kern = functools.partial(
        pl.kernel,
        out_type=[
            jax.ShapeDtypeStruct((EPAD,), jnp.float32),
            jax.ShapeDtypeStruct((EPAD,), jnp.float32),
        ],
        mesh=mesh,
        compiler_params=pltpu.CompilerParams(needs_layout_passes=False),
        scratch_types=[
            pltpu.VMEM((CHUNK_A,), jnp.int32),
            pltpu.VMEM((CHUNK_A,), jnp.int32),
            pltpu.VMEM((CHUNK_A,), jnp.float32),
            pltpu.VMEM((CHUNK_A,), jnp.float32),
            pltpu.VMEM((CHUNK_A,), jnp.float32),
            pltpu.VMEM((CHUNK_A,), jnp.float32),
            pltpu.VMEM((CHUNK_A,), jnp.float32),
            pltpu.VMEM((NPAD,), jnp.float32),
            pltpu.VMEM((NODE_TILE,), jnp.float32),
            pltpu.VMEM((NODE_TILE,), jnp.float32),
            pltpu.VMEM((16,), jnp.float32),
            pltpu.VMEM_SHARED((NPAD,), jnp.float32),
            pltpu.VMEM_SHARED((NPAD,), jnp.float32),
            pltpu.VMEM_SHARED((NPAD,), jnp.float32),
            pltpu.VMEM_SHARED((NPAD,), jnp.float32),
            pltpu.VMEM_SHARED((NSUB * HNP,), jnp.float32),
            pltpu.SemaphoreType.DMA,
            pltpu.SemaphoreType.DMA,
        ],
    )(_softmax_body)
    return kern(src, dst, ea, s, t, par)


NPASS = 8
PASS_ROWS = NPAD // NPASS
EB_TILE = EPAD // (NSUB * NC)
CHUNK_B = 1568
NCHUNK_B = EB_TILE // CHUNK_B
STAGE_B = 1664
DRAIN_W = 56
TILE_ROWS = PASS_ROWS // NSUB


def _agg_body(src_hbm, dst_hbm, coef_hbm, h_hbm, opart_hbm,
              c_src, c_dst, c_cof, st_src, st_dst, st_cof,
              blki0, blkd0, blkc0, blki1, blkd1, blkc1,
              rows0_v, rows1_v, zero_v, acc_sp, semg0, semg1):
    cid = lax.axis_index("c")
    sid = lax.axis_index("s")
    lanes = lax.iota(jnp.int32, 16)
    zero16 = jnp.zeros((16,), jnp.float32)
    wid = cid * NSUB + sid
    tbase = pl.multiple_of(wid * EB_TILE, 8)
    rb0 = sid * TILE_ROWS

    def zinit(r, _):
        for j in range(8):
            zero_v[r, pl.ds(j * 16, 16)] = zero16
        return 0
    lax.fori_loop(0, DRAIN_W, zinit, 0)

    for p in range(NPASS):
        prow_base = p * PASS_ROWS
        for w in range(7):
            rs = pl.multiple_of(rb0 + w * DRAIN_W, 8)
            pltpu.sync_copy(zero_v, acc_sp.at[pl.ds(rs, DRAIN_W)])
        plsc.subcore_barrier()

        def chunkb(k, _):
            off = pl.multiple_of(tbase + k * CHUNK_B, 8)
            pltpu.sync_copy(src_hbm.at[pl.ds(off, CHUNK_B)], c_src)
            pltpu.sync_copy(dst_hbm.at[pl.ds(off, CHUNK_B)], c_dst)
            pltpu.sync_copy(coef_hbm.at[pl.ds(off, CHUNK_B)], c_cof)

            def vstage(j, cnt):
                dl = c_dst[pl.ds(j * 16, 16)] - prow_base
                m = (dl >= 0) & (dl < PASS_ROWS)
                plsc.store_compressed(st_src.at[pl.ds(cnt, 16)],
                                      c_src[pl.ds(j * 16, 16)], mask=m)
                plsc.store_compressed(st_dst.at[pl.ds(cnt, 16)], dl, mask=m)
                plsc.store_compressed(st_cof.at[pl.ds(cnt, 16)],
                                      c_cof[pl.ds(j * 16, 16)], mask=m)
                return cnt + jnp.sum(m.astype(jnp.int32))

            cnt = lax.fori_loop(0, CHUNK_B // 16, vstage, 0)
            nblk = (cnt + 127) // 128

            def prep(b, blki, blkd, blkc, rows, semg):
                boff = b * 128
                for j in range(8):
                    pos = boff + j * 16 + lanes
                    vv = pos < cnt
                    sidx = st_src[pl.ds(boff + j * 16, 16)]
                    didx = st_dst[pl.ds(boff + j * 16, 16)]
                    cv = st_cof[pl.ds(boff + j * 16, 16)]
                    fb = j * 16 + lanes
                    blki[pl.ds(j * 16, 16)] = jnp.where(vv, sidx, fb)
                    blkd[pl.ds(j * 16, 16)] = jnp.where(vv, didx, fb)
                    blkc[pl.ds(j * 16, 16)] = jnp.where(vv, cv, zero16)
                pltpu.async_copy(h_hbm.at[blki], rows, semg)

            def finish(blki, blkd, blkc, rows, semg):
                pltpu.make_async_copy(h_hbm.at[blki], rows, semg).wait()

                def scale(r, _):
                    cvec = plsc.load_gather(
                        blkc, [jnp.full((16,), r, jnp.int32)])
                    for jj in range(8):
                        rows[r, pl.ds(jj * 16, 16)] = (
                            rows[r, pl.ds(jj * 16, 16)] * cvec)
                    return 0
                lax.fori_loop(0, 128, scale, 0)
                pltpu.sync_copy(rows, acc_sp.at[blkd], add=True)

            B0 = (blki0, blkd0, blkc0, rows0_v, semg0)
            B1 = (blki1, blkd1, blkc1, rows1_v, semg1)

            @pl.when(nblk > 0)
            def _():
                prep(0, *B0)

            def gpair(i, _):
                b1 = 2 * i + 1

                @pl.when(b1 < nblk)
                def _():
                    prep(b1, *B1)
                finish(*B0)

                @pl.when(b1 < nblk)
                def _():
                    @pl.when(b1 + 1 < nblk)
                    def _():
                        prep(b1 + 1, *B0)
                    finish(*B1)
                return 0

            lax.fori_loop(0, (nblk + 1) // 2, gpair, 0)
            return 0

        lax.fori_loop(0, NCHUNK_B, chunkb, 0)
        plsc.subcore_barrier()

        for w in range(7):
            rs = pl.multiple_of(rb0 + w * DRAIN_W, 8)
            pltpu.sync_copy(acc_sp.at[pl.ds(rs, DRAIN_W)],
                            rows0_v.at[pl.ds(0, DRAIN_W)])
            pltpu.sync_copy(
                rows0_v.at[pl.ds(0, DRAIN_W)],
                opart_hbm.at[cid, pl.ds(pl.multiple_of(prow_base, 8) + rs,
                                        DRAIN_W)])
        plsc.subcore_barrier()


def _sc_aggregate(src, dst, coef, h):
    mesh = plsc.VectorSubcoreMesh(core_axis_name="c", subcore_axis_name="s")
    kern = functools.partial(
        pl.kernel,
        out_type=[jax.ShapeDtypeStruct((NC, NPAD, D), jnp.float32)],
        mesh=mesh,
        compiler_params=pltpu.CompilerParams(needs_layout_passes=False),
        scratch_types=[
            pltpu.VMEM((CHUNK_B,), jnp.int32),
            pltpu.VMEM((CHUNK_B,), jnp.int32),
            pltpu.VMEM((CHUNK_B,), jnp.float32),
            pltpu.VMEM((STAGE_B,), jnp.int32),
            pltpu.VMEM((STAGE_B,), jnp.int32),
            pltpu.VMEM((STAGE_B,), jnp.float32),
            pltpu.VMEM((128,), jnp.int32),
            pltpu.VMEM((128,), jnp.int32),
            pltpu.VMEM((128,), jnp.float32),
            pltpu.VMEM((128,), jnp.int32),
            pltpu.VMEM((128,), jnp.int32),
            pltpu.VMEM((128,), jnp.float32),
            pltpu.VMEM((128, D), jnp.float32),
            pltpu.VMEM((128, D), jnp.float32),
            pltpu.VMEM((DRAIN_W, D), jnp.float32),
            pltpu.VMEM_SHARED((PASS_ROWS, D), jnp.float32),
            pltpu.SemaphoreType.DMA,
            pltpu.SemaphoreType.DMA,
        ],
    )(_agg_body)
    (opart,) = kern(src, dst, coef, h)
    return opart


def _combine_body(w_ref, o0_ref, o1_ref, b_ref, out_ref):
    o0 = o0_ref[0] + o0_ref[1]
    o1 = o1_ref[0] + o1_ref[1]
    out_ref[...] = w_ref[0] * o0 + w_ref[1] * o1 + b_ref[...]


def _combine(w, opart0, opart1, bias):
    grid = (N // N_NODES_BLK,)
    return pl.pallas_call(
        _combine_body,
        grid=grid,
        in_specs=[
            pl.BlockSpec(memory_space=pltpu.SMEM),
            pl.BlockSpec((NC, N_NODES_BLK, D), lambda i: (0, i, 0)),
            pl.BlockSpec((NC, N_NODES_BLK, D), lambda i: (0, i, 0)),
            pl.BlockSpec((1, D), lambda i: (0, 0)),
        ],
        out_specs=pl.BlockSpec((N_NODES_BLK, D), lambda i: (i, 0)),
        out_shape=jax.ShapeDtypeStruct((N, D), jnp.float32),
    )(w, opart0, opart1, bias.reshape(1, D))


def _attend_rel(h, s, t, edge_index, edge_attr, a_last):
    src = jnp.pad(edge_index[0], (0, EPAD - E))
    dst = jnp.pad(edge_index[1], (0, EPAD - E))
    ea1 = jnp.pad(edge_attr[:, 0], (0, EPAD - E))
    sp = jnp.pad(s, (0, NPAD - N))
    tp = jnp.pad(t, (0, NPAD - N))
    par = jnp.full((16,), a_last, jnp.float32)
    alpha_p, coef_p = _sc_softmax(src, dst, ea1, sp, tp, par)
    opart = _sc_aggregate(src, dst, coef_p, h)
    return opart, alpha_p[:E]


def kernel(x, edge_index_r0, edge_attr_r0, edge_index_r1, edge_attr_r1,
           W0, W1, a0, a1, relation_logits, bias):
    h0, h1, st = _project(x, W0, W1, a0, a1)
    op0, alpha0 = _attend_rel(h0, st[:, 0], st[:, 1], edge_index_r0,
                              edge_attr_r0, a0[2 * D])
    op1, alpha1 = _attend_rel(h1, st[:, 2], st[:, 3], edge_index_r1,
                              edge_attr_r1, a1[2 * D])
    weights = jax.nn.softmax(relation_logits, axis=0)
    out = _combine(weights, op0, op1, bias)
    return (out, alpha0, alpha1)


# speedup vs baseline: 10.2068x; 1.0426x over previous
import functools
import jax
import jax.numpy as jnp
from jax import lax
from jax.experimental import pallas as pl
from jax.experimental.pallas import tpu as pltpu
from jax.experimental.pallas import tpu_sc as plsc

N = 50000
E = 400000
D = 128
NPAD = 50176
HNP = NPAD // 2
EPAD = 401408
NSUB = 16
NC = 2
NODE_TILE = NPAD // NSUB
EA_TILE = EPAD // NSUB
CHUNK_A = 1792
NCHUNK_A = EA_TILE // CHUNK_A
GSUB = CHUNK_A // 128
EW_TILE = EPAD // (NSUB * NC)
NCHUNK_W = EW_TILE // CHUNK_A

N_NODES_BLK = 2000


def _proj_body(x_ref, w0_ref, w1_ref, a2_ref, h0_ref, h1_ref, st_ref):
    x = x_ref[...]
    h0 = jnp.dot(x, w0_ref[...], preferred_element_type=jnp.float32)
    h1 = jnp.dot(x, w1_ref[...], preferred_element_type=jnp.float32)
    h0_ref[...] = h0
    h1_ref[...] = h1
    a2 = a2_ref[...]
    st0 = jnp.dot(h0, a2[:, 0:2], preferred_element_type=jnp.float32)
    st1 = jnp.dot(h1, a2[:, 2:4], preferred_element_type=jnp.float32)
    st_ref[...] = jnp.concatenate(
        [st0, st1, jnp.zeros_like(st0), jnp.zeros_like(st0)], axis=-1)


def _project(x, W0, W1, a0, a1):
    a2 = jnp.stack([a0[:D], a0[D:2 * D], a1[:D], a1[D:2 * D]], axis=-1)
    a2 = jnp.pad(a2, ((0, 0), (0, 4)))
    grid = (N // N_NODES_BLK,)
    h0, h1, st = pl.pallas_call(
        _proj_body,
        grid=grid,
        in_specs=[
            pl.BlockSpec((N_NODES_BLK, D), lambda i: (i, 0)),
            pl.BlockSpec((D, D), lambda i: (0, 0)),
            pl.BlockSpec((D, D), lambda i: (0, 0)),
            pl.BlockSpec((D, 8), lambda i: (0, 0)),
        ],
        out_specs=[
            pl.BlockSpec((N_NODES_BLK, D), lambda i: (i, 0)),
            pl.BlockSpec((N_NODES_BLK, D), lambda i: (i, 0)),
            pl.BlockSpec((N_NODES_BLK, 8), lambda i: (i, 0)),
        ],
        out_shape=[
            jax.ShapeDtypeStruct((N, D), jnp.float32),
            jax.ShapeDtypeStruct((N, D), jnp.float32),
            jax.ShapeDtypeStruct((N, 8), jnp.float32),
        ],
    )(x, W0, W1, a2)
    return h0, h1, st


def _softmax_body(src_hbm, dst_hbm, ea_hbm, s_hbm, t_hbm, par_hbm,
                  alpha_hbm, coef_hbm,
                  idx_s, idx_d, ea_v, g1_v, g2_v, g3_v, g4_v,
                  acc_v, comb_v, tmp_v, pv_v,
                  s_sp, t_sp, emax_sp, inv_sp, red_sp, sem, sem2):
    cid = lax.axis_index("c")
    sid = lax.axis_index("s")
    lanes = lax.iota(jnp.int32, 16)
    one = jnp.ones((16,), jnp.int32)
    neg_inf = jnp.full((16,), -jnp.inf, jnp.float32)
    zero16 = jnp.zeros((16,), jnp.float32)

    obase = pl.multiple_of(sid * NODE_TILE, 8)
    pltpu.sync_copy(s_hbm.at[pl.ds(obase, NODE_TILE)], comb_v)
    pltpu.sync_copy(comb_v, s_sp.at[pl.ds(obase, NODE_TILE)])
    pltpu.sync_copy(t_hbm.at[pl.ds(obase, NODE_TILE)], comb_v)
    pltpu.sync_copy(comb_v, t_sp.at[pl.ds(obase, NODE_TILE)])
    pltpu.sync_copy(par_hbm, pv_v)

    def initm(i, _):
        acc_v[pl.ds(i * 16, 16)] = neg_inf
        return 0
    lax.fori_loop(0, NPAD // 16, initm, 0)
    plsc.subcore_barrier()

    a256 = pv_v[...]
    ebase = pl.multiple_of(sid * EA_TILE, 8)

    def seg_update(idx, val, bits0, is_max):
        def cond(b):
            return b > 0

        def step(b):
            act = ((b >> lanes) & 1) != 0
            cur = plsc.load_gather(acc_v, [idx], mask=act)
            plsc.store_scatter(acc_v, [idx], plsc.bitcast(lanes, jnp.float32),
                               mask=act)
            back = plsc.bitcast(plsc.load_gather(acc_v, [idx], mask=act),
                                jnp.int32)
            win = (back == lanes) & act
            newv = jnp.maximum(cur, val) if is_max else cur + val
            plsc.store_scatter(acc_v, [idx], newv, mask=win)
            rem = act & jnp.logical_not(win)
            return jnp.sum(jnp.where(rem, one << lanes, 0))

        lax.while_loop(cond, step, bits0)

    def gather128(table_sp, idx_ref, out_ref, s):
        descs = []
        for g in range(GSUB):
            descs.append(pltpu.async_copy(
                table_sp.at[idx_ref.at[pl.ds(g * 128, 128)]],
                out_ref.at[pl.ds(g * 128, 128)], s))
        return descs

    def load_edges(off):
        pltpu.sync_copy(src_hbm.at[pl.ds(off, CHUNK_A)], idx_s)
        pltpu.sync_copy(dst_hbm.at[pl.ds(off, CHUNK_A)], idx_d)
        pltpu.sync_copy(ea_hbm.at[pl.ds(off, CHUNK_A)], ea_v)
        ds1 = gather128(s_sp, idx_s, g1_v, sem)
        ds2 = gather128(t_sp, idx_d, g2_v, sem2)
        for d in ds1 + ds2:
            d.wait()

    def compute_e(j):
        sv = g1_v[pl.ds(j * 16, 16)]
        tv = g2_v[pl.ds(j * 16, 16)]
        eav = ea_v[pl.ds(j * 16, 16)]
        e = sv + tv + eav * a256
        return jnp.where(e >= 0, e, e * jnp.float32(0.2))

    def combine(is_max, dst_sp):
        for half in range(2):
            hbase = half * HNP
            pltpu.sync_copy(
                acc_v.at[pl.ds(hbase, HNP)],
                red_sp.at[pl.ds(pl.multiple_of(sid * HNP, 8), HNP)])
            plsc.subcore_barrier()

            @pl.when((sid // 8) == half)
            def _():
                lbase = pl.multiple_of((sid - half * 8) * NODE_TILE, 8)

                def cinit(i, _):
                    comb_v[pl.ds(i * 16, 16)] = (neg_inf if is_max
                                                 else zero16)
                    return 0
                lax.fori_loop(0, NODE_TILE // 16, cinit, 0)

                def creduce(t, _):
                    pltpu.sync_copy(
                        red_sp.at[pl.ds(
                            pl.multiple_of(t * HNP, 8) + lbase, NODE_TILE)],
                        tmp_v)

                    def vred(i, _):
                        a = comb_v[pl.ds(i * 16, 16)]
                        b = tmp_v[pl.ds(i * 16, 16)]
                        comb_v[pl.ds(i * 16, 16)] = (
                            jnp.maximum(a, b) if is_max else a + b)
                        return 0
                    lax.fori_loop(0, NODE_TILE // 16, vred, 0)
                    return 0
                lax.fori_loop(0, NSUB, creduce, 0)

                def cfin(i, _):
                    v = comb_v[pl.ds(i * 16, 16)]
                    if is_max:
                        v = jnp.where(v == neg_inf, zero16, v)
                    else:
                        v = jnp.float32(1.0) / jnp.maximum(
                            v, jnp.full((16,), 1e-12, jnp.float32))
                    comb_v[pl.ds(i * 16, 16)] = v
                    return 0
                lax.fori_loop(0, NODE_TILE // 16, cfin, 0)
                pltpu.sync_copy(comb_v, dst_sp.at[pl.ds(obase, NODE_TILE)])
            plsc.subcore_barrier()

    def chunk1(k, _):
        off = pl.multiple_of(ebase + k * CHUNK_A, 8)
        load_edges(off)

        def vloop(j, _):
            e = compute_e(j)
            idx = idx_d[pl.ds(j * 16, 16)]
            seg_update(idx, e, jnp.int32(0xFFFF), True)
            return 0

        lax.fori_loop(0, CHUNK_A // 16, vloop, 0)
        return 0

    lax.fori_loop(0, NCHUNK_A, chunk1, 0)

    combine(True, emax_sp)

    def initz(i, _):
        acc_v[pl.ds(i * 16, 16)] = zero16
        return 0
    lax.fori_loop(0, NPAD // 16, initz, 0)
    plsc.subcore_barrier()

    def chunk2(k, _):
        off = pl.multiple_of(ebase + k * CHUNK_A, 8)
        load_edges(off)
        for d in gather128(emax_sp, idx_d, g3_v, sem):
            d.wait()

        def vloop(j, _):
            e = compute_e(j)
            em = g3_v[pl.ds(j * 16, 16)]
            au = jnp.exp(e - em)
            idx = idx_d[pl.ds(j * 16, 16)]
            seg_update(idx, au, jnp.int32(0xFFFF), False)
            return 0

        lax.fori_loop(0, CHUNK_A // 16, vloop, 0)
        return 0

    lax.fori_loop(0, NCHUNK_A, chunk2, 0)

    combine(False, inv_sp)

    wbase = sid * EA_TILE + cid * EW_TILE

    def chunk3(k, _):
        off = pl.multiple_of(wbase + k * CHUNK_A, 8)
        load_edges(off)
        ds3 = gather128(emax_sp, idx_d, g3_v, sem)
        ds4 = gather128(inv_sp, idx_d, g4_v, sem2)
        for d in ds3 + ds4:
            d.wait()

        def vloop(j, _):
            e = compute_e(j)
            em = g3_v[pl.ds(j * 16, 16)]
            iv = g4_v[pl.ds(j * 16, 16)]
            eav = ea_v[pl.ds(j * 16, 16)]
            valid = (off + j * 16 + lanes) < E
            alpha = jnp.where(valid, jnp.exp(e - em) * iv, zero16)
            ew = jnp.maximum(jnp.abs(eav), jnp.full((16,), 0.01, jnp.float32))
            g1_v[pl.ds(j * 16, 16)] = alpha
            g2_v[pl.ds(j * 16, 16)] = alpha * ew
            return 0

        lax.fori_loop(0, CHUNK_A // 16, vloop, 0)
        pltpu.sync_copy(g1_v, alpha_hbm.at[pl.ds(off, CHUNK_A)])
        pltpu.sync_copy(g2_v, coef_hbm.at[pl.ds(off, CHUNK_A)])
        return 0

    lax.fori_loop(0, NCHUNK_W, chunk3, 0)


def _sc_softmax(src, dst, ea, s, t, par):
    mesh = plsc.VectorSubcoreMesh(core_axis_name="c", subcore_axis_name="s")
    kern = functools.partial(
        pl.kernel,
        out_type=[
            jax.ShapeDtypeStruct((EPAD,), jnp.float32),
            jax.ShapeDtypeStruct((EPAD,), jnp.float32),
        ],
        mesh=mesh,
        compiler_params=pltpu.CompilerParams(needs_layout_passes=False),
        scratch_types=[
            pltpu.VMEM((CHUNK_A,), jnp.int32),
            pltpu.VMEM((CHUNK_A,), jnp.int32),
            pltpu.VMEM((CHUNK_A,), jnp.float32),
            pltpu.VMEM((CHUNK_A,), jnp.float32),
            pltpu.VMEM((CHUNK_A,), jnp.float32),
            pltpu.VMEM((CHUNK_A,), jnp.float32),
            pltpu.VMEM((CHUNK_A,), jnp.float32),
            pltpu.VMEM((NPAD,), jnp.float32),
            pltpu.VMEM((NODE_TILE,), jnp.float32),
            pltpu.VMEM((NODE_TILE,), jnp.float32),
            pltpu.VMEM((16,), jnp.float32),
            pltpu.VMEM_SHARED((NPAD,), jnp.float32),
            pltpu.VMEM_SHARED((NPAD,), jnp.float32),
            pltpu.VMEM_SHARED((NPAD,), jnp.float32),
            pltpu.VMEM_SHARED((NPAD,), jnp.float32),
            pltpu.VMEM_SHARED((NSUB * HNP,), jnp.float32),
            pltpu.SemaphoreType.DMA,
            pltpu.SemaphoreType.DMA,
        ],
    )(_softmax_body)
    return kern(src, dst, ea, s, t, par)


NPASS = 8
PASS_ROWS = NPAD // NPASS
EB_TILE = EPAD // (NSUB * NC)
CHUNK_B = 1568
NCHUNK_B = EB_TILE // CHUNK_B
STAGE_B = 1664
DRAIN_W = 56
TILE_ROWS = PASS_ROWS // NSUB


def _agg_body(src_hbm, dst_hbm, coef_hbm, h_hbm, opart_hbm,
              c_src, c_dst, c_cof, st_src, st_dst, st_cof,
              blki0, blkd0, blkc0, blki1, blkd1, blkc1,
              rows0_v, rows1_v, zero_v, acc_sp, semg0, semg1):
    cid = lax.axis_index("c")
    sid = lax.axis_index("s")
    lanes = lax.iota(jnp.int32, 16)
    zero16 = jnp.zeros((16,), jnp.float32)
    wid = cid * NSUB + sid
    tbase = pl.multiple_of(wid * EB_TILE, 8)
    rb0 = sid * TILE_ROWS

    def zinit(r, _):
        for j in range(8):
            zero_v[r, pl.ds(j * 16, 16)] = zero16
        return 0
    lax.fori_loop(0, DRAIN_W, zinit, 0)

    for p in range(NPASS):
        prow_base = p * PASS_ROWS
        for w in range(7):
            rs = pl.multiple_of(rb0 + w * DRAIN_W, 8)
            pltpu.sync_copy(zero_v, acc_sp.at[pl.ds(rs, DRAIN_W)])
        plsc.subcore_barrier()

        def chunkb(k, _):
            off = pl.multiple_of(tbase + k * CHUNK_B, 8)
            pltpu.sync_copy(src_hbm.at[pl.ds(off, CHUNK_B)], c_src)
            pltpu.sync_copy(dst_hbm.at[pl.ds(off, CHUNK_B)], c_dst)
            pltpu.sync_copy(coef_hbm.at[pl.ds(off, CHUNK_B)], c_cof)

            def vstage(j, cnt):
                dl = c_dst[pl.ds(j * 16, 16)] - prow_base
                m = (dl >= 0) & (dl < PASS_ROWS)
                plsc.store_compressed(st_src.at[pl.ds(cnt, 16)],
                                      c_src[pl.ds(j * 16, 16)], mask=m)
                plsc.store_compressed(st_dst.at[pl.ds(cnt, 16)], dl, mask=m)
                plsc.store_compressed(st_cof.at[pl.ds(cnt, 16)],
                                      c_cof[pl.ds(j * 16, 16)], mask=m)
                return cnt + jnp.sum(m.astype(jnp.int32))

            cnt = lax.fori_loop(0, CHUNK_B // 16, vstage, 0)
            nblk = (cnt + 127) // 128

            def prep(b, blki, blkd, blkc, rows, semg):
                boff = b * 128
                for j in range(8):
                    pos = boff + j * 16 + lanes
                    vv = pos < cnt
                    sidx = st_src[pl.ds(boff + j * 16, 16)]
                    didx = st_dst[pl.ds(boff + j * 16, 16)]
                    cv = st_cof[pl.ds(boff + j * 16, 16)]
                    fb = j * 16 + lanes
                    blki[pl.ds(j * 16, 16)] = jnp.where(vv, sidx, fb)
                    blkd[pl.ds(j * 16, 16)] = jnp.where(
                        vv, didx, jnp.full((16,), PASS_ROWS, jnp.int32))
                    blkc[pl.ds(j * 16, 16)] = cv
                pltpu.async_copy(h_hbm.at[blki], rows, semg)

            def finish(b, blki, blkd, blkc, rows, semg):
                pltpu.make_async_copy(h_hbm.at[blki], rows, semg).wait()
                nvalid = jnp.minimum(cnt - b * 128, 128)

                @plsc.parallel_loop(0, nvalid, unroll=4)
                def scale(r):
                    cvec = plsc.load_gather(
                        blkc, [jnp.full((16,), r, jnp.int32)])
                    for jj in range(8):
                        rows[r, pl.ds(jj * 16, 16)] = (
                            rows[r, pl.ds(jj * 16, 16)] * cvec)
                pltpu.sync_copy(rows, acc_sp.at[blkd], add=True)

            B0 = (blki0, blkd0, blkc0, rows0_v, semg0)
            B1 = (blki1, blkd1, blkc1, rows1_v, semg1)

            @pl.when(nblk > 0)
            def _():
                prep(0, *B0)

            def gpair(i, _):
                b1 = 2 * i + 1

                @pl.when(b1 < nblk)
                def _():
                    prep(b1, *B1)
                finish(b1 - 1, *B0)

                @pl.when(b1 < nblk)
                def _():
                    @pl.when(b1 + 1 < nblk)
                    def _():
                        prep(b1 + 1, *B0)
                    finish(b1, *B1)
                return 0

            lax.fori_loop(0, (nblk + 1) // 2, gpair, 0)
            return 0

        lax.fori_loop(0, NCHUNK_B, chunkb, 0)
        plsc.subcore_barrier()

        for w in range(7):
            rs = pl.multiple_of(rb0 + w * DRAIN_W, 8)
            pltpu.sync_copy(acc_sp.at[pl.ds(rs, DRAIN_W)],
                            rows0_v.at[pl.ds(0, DRAIN_W)])
            pltpu.sync_copy(
                rows0_v.at[pl.ds(0, DRAIN_W)],
                opart_hbm.at[cid, pl.ds(pl.multiple_of(prow_base, 8) + rs,
                                        DRAIN_W)])
        plsc.subcore_barrier()


def _sc_aggregate(src, dst, coef, h):
    mesh = plsc.VectorSubcoreMesh(core_axis_name="c", subcore_axis_name="s")
    kern = functools.partial(
        pl.kernel,
        out_type=[jax.ShapeDtypeStruct((NC, NPAD, D), jnp.float32)],
        mesh=mesh,
        compiler_params=pltpu.CompilerParams(needs_layout_passes=False),
        scratch_types=[
            pltpu.VMEM((CHUNK_B,), jnp.int32),
            pltpu.VMEM((CHUNK_B,), jnp.int32),
            pltpu.VMEM((CHUNK_B,), jnp.float32),
            pltpu.VMEM((STAGE_B,), jnp.int32),
            pltpu.VMEM((STAGE_B,), jnp.int32),
            pltpu.VMEM((STAGE_B,), jnp.float32),
            pltpu.VMEM((128,), jnp.int32),
            pltpu.VMEM((128,), jnp.int32),
            pltpu.VMEM((128,), jnp.float32),
            pltpu.VMEM((128,), jnp.int32),
            pltpu.VMEM((128,), jnp.int32),
            pltpu.VMEM((128,), jnp.float32),
            pltpu.VMEM((128, D), jnp.float32),
            pltpu.VMEM((128, D), jnp.float32),
            pltpu.VMEM((DRAIN_W, D), jnp.float32),
            pltpu.VMEM_SHARED((PASS_ROWS + 8, D), jnp.float32),
            pltpu.SemaphoreType.DMA,
            pltpu.SemaphoreType.DMA,
        ],
    )(_agg_body)
    (opart,) = kern(src, dst, coef, h)
    return opart


def _combine_body(w_ref, o0_ref, o1_ref, b_ref, out_ref):
    o0 = o0_ref[0] + o0_ref[1]
    o1 = o1_ref[0] + o1_ref[1]
    out_ref[...] = w_ref[0] * o0 + w_ref[1] * o1 + b_ref[...]


def _combine(w, opart0, opart1, bias):
    grid = (N // N_NODES_BLK,)
    return pl.pallas_call(
        _combine_body,
        grid=grid,
        in_specs=[
            pl.BlockSpec(memory_space=pltpu.SMEM),
            pl.BlockSpec((NC, N_NODES_BLK, D), lambda i: (0, i, 0)),
            pl.BlockSpec((NC, N_NODES_BLK, D), lambda i: (0, i, 0)),
            pl.BlockSpec((1, D), lambda i: (0, 0)),
        ],
        out_specs=pl.BlockSpec((N_NODES_BLK, D), lambda i: (i, 0)),
        out_shape=jax.ShapeDtypeStruct((N, D), jnp.float32),
    )(w, opart0, opart1, bias.reshape(1, D))


def _attend_rel(h, s, t, edge_index, edge_attr, a_last):
    src = jnp.pad(edge_index[0], (0, EPAD - E))
    dst = jnp.pad(edge_index[1], (0, EPAD - E), constant_values=NPAD - 8)
    ea1 = jnp.pad(edge_attr[:, 0], (0, EPAD - E))
    sp = jnp.pad(s, (0, NPAD - N))
    tp = jnp.pad(t, (0, NPAD - N))
    par = jnp.full((16,), a_last, jnp.float32)
    alpha_p, coef_p = _sc_softmax(src, dst, ea1, sp, tp, par)
    opart = _sc_aggregate(src, dst, coef_p, h)
    return opart, alpha_p[:E]


def kernel(x, edge_index_r0, edge_attr_r0, edge_index_r1, edge_attr_r1,
           W0, W1, a0, a1, relation_logits, bias):
    h0, h1, st = _project(x, W0, W1, a0, a1)
    op0, alpha0 = _attend_rel(h0, st[:, 0], st[:, 1], edge_index_r0,
                              edge_attr_r0, a0[2 * D])
    op1, alpha1 = _attend_rel(h1, st[:, 2], st[:, 3], edge_index_r1,
                              edge_attr_r1, a1[2 * D])
    weights = jax.nn.softmax(relation_logits, axis=0)
    out = _combine(weights, op0, op1, bias)
    return (out, alpha0, alpha1)


# speedup vs baseline: 10.8230x; 1.0604x over previous
import functools
import jax
import jax.numpy as jnp
from jax import lax
from jax.experimental import pallas as pl
from jax.experimental.pallas import tpu as pltpu
from jax.experimental.pallas import tpu_sc as plsc

N = 50000
E = 400000
D = 128
NPAD = 50176
HNP = NPAD // 2
EPAD = 401408
NSUB = 16
NC = 2
NODE_TILE = NPAD // NSUB
EA_TILE = EPAD // NSUB
CHUNK_A = 1792
NCHUNK_A = EA_TILE // CHUNK_A
GSUB = CHUNK_A // 128
EW_TILE = EPAD // (NSUB * NC)
NCHUNK_W = EW_TILE // CHUNK_A

N_NODES_BLK = 2000


def _proj_body(x_ref, w0_ref, w1_ref, a2_ref, h0_ref, h1_ref, st_ref):
    x = x_ref[...]
    h0 = jnp.dot(x, w0_ref[...], preferred_element_type=jnp.float32)
    h1 = jnp.dot(x, w1_ref[...], preferred_element_type=jnp.float32)
    h0_ref[...] = h0
    h1_ref[...] = h1
    a2 = a2_ref[...]
    st0 = jnp.dot(h0, a2[:, 0:2], preferred_element_type=jnp.float32)
    st1 = jnp.dot(h1, a2[:, 2:4], preferred_element_type=jnp.float32)
    st_ref[...] = jnp.concatenate(
        [st0, st1, jnp.zeros_like(st0), jnp.zeros_like(st0)], axis=-1)


def _project(x, W0, W1, a0, a1):
    a2 = jnp.stack([a0[:D], a0[D:2 * D], a1[:D], a1[D:2 * D]], axis=-1)
    a2 = jnp.pad(a2, ((0, 0), (0, 4)))
    grid = (N // N_NODES_BLK,)
    h0, h1, st = pl.pallas_call(
        _proj_body,
        grid=grid,
        in_specs=[
            pl.BlockSpec((N_NODES_BLK, D), lambda i: (i, 0)),
            pl.BlockSpec((D, D), lambda i: (0, 0)),
            pl.BlockSpec((D, D), lambda i: (0, 0)),
            pl.BlockSpec((D, 8), lambda i: (0, 0)),
        ],
        out_specs=[
            pl.BlockSpec((N_NODES_BLK, D), lambda i: (i, 0)),
            pl.BlockSpec((N_NODES_BLK, D), lambda i: (i, 0)),
            pl.BlockSpec((N_NODES_BLK, 8), lambda i: (i, 0)),
        ],
        out_shape=[
            jax.ShapeDtypeStruct((N, D), jnp.float32),
            jax.ShapeDtypeStruct((N, D), jnp.float32),
            jax.ShapeDtypeStruct((N, 8), jnp.float32),
        ],
    )(x, W0, W1, a2)
    return h0, h1, st


def _softmax_body(src_hbm, dst_hbm, ea_hbm, s_hbm, t_hbm, par_hbm,
                  alpha_hbm, coef_hbm,
                  idx_s, idx_d, ea_v, g1_v, g2_v, g3_v, g4_v,
                  acc_v, comb_v, tmp_v, pv_v,
                  s_sp, t_sp, emax_sp, inv_sp, red_sp, sem, sem2):
    cid = lax.axis_index("c")
    sid = lax.axis_index("s")
    lanes = lax.iota(jnp.int32, 16)
    one = jnp.ones((16,), jnp.int32)
    neg_inf = jnp.full((16,), -jnp.inf, jnp.float32)
    zero16 = jnp.zeros((16,), jnp.float32)

    obase = pl.multiple_of(sid * NODE_TILE, 8)
    pltpu.sync_copy(s_hbm.at[pl.ds(obase, NODE_TILE)], comb_v)
    pltpu.sync_copy(comb_v, s_sp.at[pl.ds(obase, NODE_TILE)])
    pltpu.sync_copy(t_hbm.at[pl.ds(obase, NODE_TILE)], comb_v)
    pltpu.sync_copy(comb_v, t_sp.at[pl.ds(obase, NODE_TILE)])
    pltpu.sync_copy(par_hbm, pv_v)

    def initm(i, _):
        acc_v[pl.ds(i * 16, 16)] = neg_inf
        return 0
    lax.fori_loop(0, NPAD // 16, initm, 0)
    plsc.subcore_barrier()

    a256 = pv_v[...]
    ebase = pl.multiple_of(sid * EA_TILE, 8)

    def seg_update(idx, val, bits0, is_max):
        def cond(b):
            return b > 0

        def step(b):
            act = ((b >> lanes) & 1) != 0
            cur = plsc.load_gather(acc_v, [idx], mask=act)
            plsc.store_scatter(acc_v, [idx], plsc.bitcast(lanes, jnp.float32),
                               mask=act)
            back = plsc.bitcast(plsc.load_gather(acc_v, [idx], mask=act),
                                jnp.int32)
            win = (back == lanes) & act
            newv = jnp.maximum(cur, val) if is_max else cur + val
            plsc.store_scatter(acc_v, [idx], newv, mask=win)
            rem = act & jnp.logical_not(win)
            return jnp.sum(jnp.where(rem, one << lanes, 0))

        lax.while_loop(cond, step, bits0)

    def gather128(table_sp, idx_ref, out_ref, s):
        descs = []
        for g in range(GSUB):
            descs.append(pltpu.async_copy(
                table_sp.at[idx_ref.at[pl.ds(g * 128, 128)]],
                out_ref.at[pl.ds(g * 128, 128)], s))
        return descs

    def load_edges(off):
        pltpu.sync_copy(src_hbm.at[pl.ds(off, CHUNK_A)], idx_s)
        pltpu.sync_copy(dst_hbm.at[pl.ds(off, CHUNK_A)], idx_d)
        pltpu.sync_copy(ea_hbm.at[pl.ds(off, CHUNK_A)], ea_v)
        ds1 = gather128(s_sp, idx_s, g1_v, sem)
        ds2 = gather128(t_sp, idx_d, g2_v, sem2)
        for d in ds1 + ds2:
            d.wait()

    def compute_e(j):
        sv = g1_v[pl.ds(j * 16, 16)]
        tv = g2_v[pl.ds(j * 16, 16)]
        eav = ea_v[pl.ds(j * 16, 16)]
        e = sv + tv + eav * a256
        return jnp.where(e >= 0, e, e * jnp.float32(0.2))

    def combine(is_max, dst_sp):
        for half in range(2):
            hbase = half * HNP
            pltpu.sync_copy(
                acc_v.at[pl.ds(hbase, HNP)],
                red_sp.at[pl.ds(pl.multiple_of(sid * HNP, 8), HNP)])
            plsc.subcore_barrier()

            @pl.when((sid // 8) == half)
            def _():
                lbase = pl.multiple_of((sid - half * 8) * NODE_TILE, 8)

                def cinit(i, _):
                    comb_v[pl.ds(i * 16, 16)] = (neg_inf if is_max
                                                 else zero16)
                    return 0
                lax.fori_loop(0, NODE_TILE // 16, cinit, 0)

                def creduce(t, _):
                    pltpu.sync_copy(
                        red_sp.at[pl.ds(
                            pl.multiple_of(t * HNP, 8) + lbase, NODE_TILE)],
                        tmp_v)

                    def vred(i, _):
                        a = comb_v[pl.ds(i * 16, 16)]
                        b = tmp_v[pl.ds(i * 16, 16)]
                        comb_v[pl.ds(i * 16, 16)] = (
                            jnp.maximum(a, b) if is_max else a + b)
                        return 0
                    lax.fori_loop(0, NODE_TILE // 16, vred, 0)
                    return 0
                lax.fori_loop(0, NSUB, creduce, 0)

                def cfin(i, _):
                    v = comb_v[pl.ds(i * 16, 16)]
                    if is_max:
                        v = jnp.where(v == neg_inf, zero16, v)
                    else:
                        v = jnp.float32(1.0) / jnp.maximum(
                            v, jnp.full((16,), 1e-12, jnp.float32))
                    comb_v[pl.ds(i * 16, 16)] = v
                    return 0
                lax.fori_loop(0, NODE_TILE // 16, cfin, 0)
                pltpu.sync_copy(comb_v, dst_sp.at[pl.ds(obase, NODE_TILE)])
            plsc.subcore_barrier()

    def chunk1(k, _):
        off = pl.multiple_of(ebase + k * CHUNK_A, 8)
        load_edges(off)

        def vloop(j, _):
            e = compute_e(j)
            idx = idx_d[pl.ds(j * 16, 16)]
            seg_update(idx, e, jnp.int32(0xFFFF), True)
            return 0

        lax.fori_loop(0, CHUNK_A // 16, vloop, 0)
        return 0

    lax.fori_loop(0, NCHUNK_A, chunk1, 0)

    combine(True, emax_sp)

    def initz(i, _):
        acc_v[pl.ds(i * 16, 16)] = zero16
        return 0
    lax.fori_loop(0, NPAD // 16, initz, 0)
    plsc.subcore_barrier()

    def chunk2(k, _):
        off = pl.multiple_of(ebase + k * CHUNK_A, 8)
        load_edges(off)
        for d in gather128(emax_sp, idx_d, g3_v, sem):
            d.wait()

        def vloop(j, _):
            e = compute_e(j)
            em = g3_v[pl.ds(j * 16, 16)]
            au = jnp.exp(e - em)
            idx = idx_d[pl.ds(j * 16, 16)]
            seg_update(idx, au, jnp.int32(0xFFFF), False)
            return 0

        lax.fori_loop(0, CHUNK_A // 16, vloop, 0)
        return 0

    lax.fori_loop(0, NCHUNK_A, chunk2, 0)

    combine(False, inv_sp)

    wbase = sid * EA_TILE + cid * EW_TILE

    def chunk3(k, _):
        off = pl.multiple_of(wbase + k * CHUNK_A, 8)
        load_edges(off)
        ds3 = gather128(emax_sp, idx_d, g3_v, sem)
        ds4 = gather128(inv_sp, idx_d, g4_v, sem2)
        for d in ds3 + ds4:
            d.wait()

        def vloop(j, _):
            e = compute_e(j)
            em = g3_v[pl.ds(j * 16, 16)]
            iv = g4_v[pl.ds(j * 16, 16)]
            eav = ea_v[pl.ds(j * 16, 16)]
            valid = (off + j * 16 + lanes) < E
            alpha = jnp.where(valid, jnp.exp(e - em) * iv, zero16)
            ew = jnp.maximum(jnp.abs(eav), jnp.full((16,), 0.01, jnp.float32))
            g1_v[pl.ds(j * 16, 16)] = alpha
            g2_v[pl.ds(j * 16, 16)] = alpha * ew
            return 0

        lax.fori_loop(0, CHUNK_A // 16, vloop, 0)
        pltpu.sync_copy(g1_v, alpha_hbm.at[pl.ds(off, CHUNK_A)])
        pltpu.sync_copy(g2_v, coef_hbm.at[pl.ds(off, CHUNK_A)])
        return 0

    lax.fori_loop(0, NCHUNK_W, chunk3, 0)


def _sc_softmax(src, dst, ea, s, t, par):
    mesh = plsc.VectorSubcoreMesh(core_axis_name="c", subcore_axis_name="s")
    kern = functools.partial(
        pl.kernel,
        out_type=[
            jax.ShapeDtypeStruct((EPAD,), jnp.float32),
            jax.ShapeDtypeStruct((EPAD,), jnp.float32),
        ],
        mesh=mesh,
        compiler_params=pltpu.CompilerParams(needs_layout_passes=False),
        scratch_types=[
            pltpu.VMEM((CHUNK_A,), jnp.int32),
            pltpu.VMEM((CHUNK_A,), jnp.int32),
            pltpu.VMEM((CHUNK_A,), jnp.float32),
            pltpu.VMEM((CHUNK_A,), jnp.float32),
            pltpu.VMEM((CHUNK_A,), jnp.float32),
            pltpu.VMEM((CHUNK_A,), jnp.float32),
            pltpu.VMEM((CHUNK_A,), jnp.float32),
            pltpu.VMEM((NPAD,), jnp.float32),
            pltpu.VMEM((NODE_TILE,), jnp.float32),
            pltpu.VMEM((NODE_TILE,), jnp.float32),
            pltpu.VMEM((16,), jnp.float32),
            pltpu.VMEM_SHARED((NPAD,), jnp.float32),
            pltpu.VMEM_SHARED((NPAD,), jnp.float32),
            pltpu.VMEM_SHARED((NPAD,), jnp.float32),
            pltpu.VMEM_SHARED((NPAD,), jnp.float32),
            pltpu.VMEM_SHARED((NSUB * HNP,), jnp.float32),
            pltpu.SemaphoreType.DMA,
            pltpu.SemaphoreType.DMA,
        ],
    )(_softmax_body)
    return kern(src, dst, ea, s, t, par)


NPASS = 8
PASS_ROWS = NPAD // NPASS
EB_TILE = EPAD // (NSUB * NC)
CHUNK_B = 1568
NCHUNK_B = EB_TILE // CHUNK_B
STAGE_B = 1664
DRAIN_W = 56
TILE_ROWS = PASS_ROWS // NSUB


def _agg_body(src_hbm, dst_hbm, coef_hbm, h_hbm, opart_hbm,
              c_src, c_dst, c_cof, st_src, st_dst, st_cof,
              blki0, blkd0, blkc0, blki1, blkd1, blkc1,
              rows0_v, rows1_v, zero_v, acc_sp, semg0, semg1, sems0, sems1):
    cid = lax.axis_index("c")
    sid = lax.axis_index("s")
    lanes = lax.iota(jnp.int32, 16)
    zero16 = jnp.zeros((16,), jnp.float32)
    wid = cid * NSUB + sid
    tbase = pl.multiple_of(wid * EB_TILE, 8)
    rb0 = sid * TILE_ROWS

    def zinit(r, _):
        for j in range(8):
            zero_v[r, pl.ds(j * 16, 16)] = zero16
        return 0
    lax.fori_loop(0, DRAIN_W, zinit, 0)

    for p in range(NPASS):
        prow_base = p * PASS_ROWS
        for w in range(7):
            rs = pl.multiple_of(rb0 + w * DRAIN_W, 8)
            pltpu.sync_copy(zero_v, acc_sp.at[pl.ds(rs, DRAIN_W)])
        plsc.subcore_barrier()

        def chunkb(k, _):
            off = pl.multiple_of(tbase + k * CHUNK_B, 8)
            pltpu.sync_copy(src_hbm.at[pl.ds(off, CHUNK_B)], c_src)
            pltpu.sync_copy(dst_hbm.at[pl.ds(off, CHUNK_B)], c_dst)
            pltpu.sync_copy(coef_hbm.at[pl.ds(off, CHUNK_B)], c_cof)

            @plsc.parallel_loop(0, CHUNK_B // 16, unroll=4, carry=jnp.int32(0))
            def vstage(j, cnt):
                dl = c_dst[pl.ds(j * 16, 16)] - prow_base
                m = (dl >= 0) & (dl < PASS_ROWS)
                plsc.store_compressed(st_src.at[pl.ds(cnt, 16)],
                                      c_src[pl.ds(j * 16, 16)], mask=m)
                plsc.store_compressed(st_dst.at[pl.ds(cnt, 16)], dl, mask=m)
                plsc.store_compressed(st_cof.at[pl.ds(cnt, 16)],
                                      c_cof[pl.ds(j * 16, 16)], mask=m)
                return cnt + jnp.sum(m.astype(jnp.int32))

            cnt = vstage
            nblk = (cnt + 127) // 128

            def prep(b, blki, blkd, blkc, rows, semg, sems):
                @pl.when(b >= 2)
                def _():
                    pltpu.make_async_copy(rows, acc_sp.at[blkd], sems).wait()
                boff = b * 128
                for j in range(8):
                    pos = boff + j * 16 + lanes
                    vv = pos < cnt
                    sidx = st_src[pl.ds(boff + j * 16, 16)]
                    didx = st_dst[pl.ds(boff + j * 16, 16)]
                    cv = st_cof[pl.ds(boff + j * 16, 16)]
                    fb = j * 16 + lanes
                    blki[pl.ds(j * 16, 16)] = jnp.where(vv, sidx, fb)
                    blkd[pl.ds(j * 16, 16)] = jnp.where(
                        vv, didx, jnp.full((16,), PASS_ROWS, jnp.int32))
                    blkc[pl.ds(j * 16, 16)] = cv
                pltpu.async_copy(h_hbm.at[blki], rows, semg)

            def finish(b, blki, blkd, blkc, rows, semg, sems):
                pltpu.make_async_copy(h_hbm.at[blki], rows, semg).wait()
                nvalid = jnp.minimum(cnt - b * 128, 128)

                @plsc.parallel_loop(0, nvalid, unroll=4)
                def scale(r):
                    cvec = plsc.load_gather(
                        blkc, [jnp.full((16,), r, jnp.int32)])
                    for jj in range(8):
                        rows[r, pl.ds(jj * 16, 16)] = (
                            rows[r, pl.ds(jj * 16, 16)] * cvec)
                pltpu.async_copy(rows, acc_sp.at[blkd], sems, add=True)

            B0 = (blki0, blkd0, blkc0, rows0_v, semg0, sems0)
            B1 = (blki1, blkd1, blkc1, rows1_v, semg1, sems1)

            @pl.when(nblk > 0)
            def _():
                prep(0, *B0)

            def gpair(i, _):
                b1 = 2 * i + 1

                @pl.when(b1 < nblk)
                def _():
                    prep(b1, *B1)
                finish(b1 - 1, *B0)

                @pl.when(b1 < nblk)
                def _():
                    @pl.when(b1 + 1 < nblk)
                    def _():
                        prep(b1 + 1, *B0)
                    finish(b1, *B1)
                return 0

            lax.fori_loop(0, (nblk + 1) // 2, gpair, 0)

            @pl.when(nblk >= 2)
            def _():
                pltpu.make_async_copy(rows0_v, acc_sp.at[blkd0], sems0).wait()
                pltpu.make_async_copy(rows1_v, acc_sp.at[blkd1], sems1).wait()

            @pl.when(nblk == 1)
            def _():
                pltpu.make_async_copy(rows0_v, acc_sp.at[blkd0], sems0).wait()
            return 0

        lax.fori_loop(0, NCHUNK_B, chunkb, 0)
        plsc.subcore_barrier()

        for w in range(7):
            rs = pl.multiple_of(rb0 + w * DRAIN_W, 8)
            pltpu.sync_copy(acc_sp.at[pl.ds(rs, DRAIN_W)],
                            rows0_v.at[pl.ds(0, DRAIN_W)])
            pltpu.sync_copy(
                rows0_v.at[pl.ds(0, DRAIN_W)],
                opart_hbm.at[cid, pl.ds(pl.multiple_of(prow_base, 8) + rs,
                                        DRAIN_W)])
        plsc.subcore_barrier()


def _sc_aggregate(src, dst, coef, h):
    mesh = plsc.VectorSubcoreMesh(core_axis_name="c", subcore_axis_name="s")
    kern = functools.partial(
        pl.kernel,
        out_type=[jax.ShapeDtypeStruct((NC, NPAD, D), jnp.float32)],
        mesh=mesh,
        compiler_params=pltpu.CompilerParams(needs_layout_passes=False),
        scratch_types=[
            pltpu.VMEM((CHUNK_B,), jnp.int32),
            pltpu.VMEM((CHUNK_B,), jnp.int32),
            pltpu.VMEM((CHUNK_B,), jnp.float32),
            pltpu.VMEM((STAGE_B,), jnp.int32),
            pltpu.VMEM((STAGE_B,), jnp.int32),
            pltpu.VMEM((STAGE_B,), jnp.float32),
            pltpu.VMEM((128,), jnp.int32),
            pltpu.VMEM((128,), jnp.int32),
            pltpu.VMEM((128,), jnp.float32),
            pltpu.VMEM((128,), jnp.int32),
            pltpu.VMEM((128,), jnp.int32),
            pltpu.VMEM((128,), jnp.float32),
            pltpu.VMEM((128, D), jnp.float32),
            pltpu.VMEM((128, D), jnp.float32),
            pltpu.VMEM((DRAIN_W, D), jnp.float32),
            pltpu.VMEM_SHARED((PASS_ROWS + 8, D), jnp.float32),
            pltpu.SemaphoreType.DMA,
            pltpu.SemaphoreType.DMA,
            pltpu.SemaphoreType.DMA,
            pltpu.SemaphoreType.DMA,
        ],
    )(_agg_body)
    (opart,) = kern(src, dst, coef, h)
    return opart


def _combine_body(w_ref, o0_ref, o1_ref, b_ref, out_ref):
    o0 = o0_ref[0] + o0_ref[1]
    o1 = o1_ref[0] + o1_ref[1]
    out_ref[...] = w_ref[0] * o0 + w_ref[1] * o1 + b_ref[...]


def _combine(w, opart0, opart1, bias):
    grid = (N // N_NODES_BLK,)
    return pl.pallas_call(
        _combine_body,
        grid=grid,
        in_specs=[
            pl.BlockSpec(memory_space=pltpu.SMEM),
            pl.BlockSpec((NC, N_NODES_BLK, D), lambda i: (0, i, 0)),
            pl.BlockSpec((NC, N_NODES_BLK, D), lambda i: (0, i, 0)),
            pl.BlockSpec((1, D), lambda i: (0, 0)),
        ],
        out_specs=pl.BlockSpec((N_NODES_BLK, D), lambda i: (i, 0)),
        out_shape=jax.ShapeDtypeStruct((N, D), jnp.float32),
    )(w, opart0, opart1, bias.reshape(1, D))


def _attend_rel(h, s, t, edge_index, edge_attr, a_last):
    src = jnp.pad(edge_index[0], (0, EPAD - E))
    dst = jnp.pad(edge_index[1], (0, EPAD - E), constant_values=NPAD - 8)
    ea1 = jnp.pad(edge_attr[:, 0], (0, EPAD - E))
    sp = jnp.pad(s, (0, NPAD - N))
    tp = jnp.pad(t, (0, NPAD - N))
    par = jnp.full((16,), a_last, jnp.float32)
    alpha_p, coef_p = _sc_softmax(src, dst, ea1, sp, tp, par)
    opart = _sc_aggregate(src, dst, coef_p, h)
    return opart, alpha_p[:E]


def kernel(x, edge_index_r0, edge_attr_r0, edge_index_r1, edge_attr_r1,
           W0, W1, a0, a1, relation_logits, bias):
    h0, h1, st = _project(x, W0, W1, a0, a1)
    op0, alpha0 = _attend_rel(h0, st[:, 0], st[:, 1], edge_index_r0,
                              edge_attr_r0, a0[2 * D])
    op1, alpha1 = _attend_rel(h1, st[:, 2], st[:, 3], edge_index_r1,
                              edge_attr_r1, a1[2 * D])
    weights = jax.nn.softmax(relation_logits, axis=0)
    out = _combine(weights, op0, op1, bias)
    return (out, alpha0, alpha1)


# speedup vs baseline: 11.3692x; 1.0505x over previous
import functools
import jax
import jax.numpy as jnp
from jax import lax
from jax.experimental import pallas as pl
from jax.experimental.pallas import tpu as pltpu
from jax.experimental.pallas import tpu_sc as plsc

N = 50000
E = 400000
D = 128
NPAD = 50176
HNP = NPAD // 2
EPAD = 401408
NSUB = 16
NC = 2
NODE_TILE = NPAD // NSUB
EA_TILE = EPAD // NSUB
CHUNK_A = 1792
NCHUNK_A = EA_TILE // CHUNK_A
GSUB = CHUNK_A // 128
EW_TILE = EPAD // (NSUB * NC)
NCHUNK_W = EW_TILE // CHUNK_A

N_NODES_BLK = 2000


def _proj_body(x_ref, w0_ref, w1_ref, a2_ref, h0_ref, h1_ref, st_ref):
    x = x_ref[...]
    h0 = jnp.dot(x, w0_ref[...], preferred_element_type=jnp.float32)
    h1 = jnp.dot(x, w1_ref[...], preferred_element_type=jnp.float32)
    h0_ref[...] = h0
    h1_ref[...] = h1
    a2 = a2_ref[...]
    st0 = jnp.dot(h0, a2[:, 0:2], preferred_element_type=jnp.float32)
    st1 = jnp.dot(h1, a2[:, 2:4], preferred_element_type=jnp.float32)
    st_ref[...] = jnp.concatenate(
        [st0, st1, jnp.zeros_like(st0), jnp.zeros_like(st0)], axis=-1)


def _project(x, W0, W1, a0, a1):
    a2 = jnp.stack([a0[:D], a0[D:2 * D], a1[:D], a1[D:2 * D]], axis=-1)
    a2 = jnp.pad(a2, ((0, 0), (0, 4)))
    grid = (N // N_NODES_BLK,)
    h0, h1, st = pl.pallas_call(
        _proj_body,
        grid=grid,
        in_specs=[
            pl.BlockSpec((N_NODES_BLK, D), lambda i: (i, 0)),
            pl.BlockSpec((D, D), lambda i: (0, 0)),
            pl.BlockSpec((D, D), lambda i: (0, 0)),
            pl.BlockSpec((D, 8), lambda i: (0, 0)),
        ],
        out_specs=[
            pl.BlockSpec((N_NODES_BLK, D), lambda i: (i, 0)),
            pl.BlockSpec((N_NODES_BLK, D), lambda i: (i, 0)),
            pl.BlockSpec((N_NODES_BLK, 8), lambda i: (i, 0)),
        ],
        out_shape=[
            jax.ShapeDtypeStruct((N, D), jnp.float32),
            jax.ShapeDtypeStruct((N, D), jnp.float32),
            jax.ShapeDtypeStruct((N, 8), jnp.float32),
        ],
    )(x, W0, W1, a2)
    return h0, h1, st


def _softmax_body(src_hbm, dst_hbm, ea_hbm, s_hbm, t_hbm, par_hbm,
                  alpha_hbm, coef_hbm,
                  idx_s, idx_d, ea_v, g1_v, g2_v, g3_v, g4_v,
                  acc_v, comb_v, tmp_v, pv_v,
                  s_sp, t_sp, emax_sp, inv_sp, red_sp, sem, sem2):
    cid = lax.axis_index("c")
    sid = lax.axis_index("s")
    lanes = lax.iota(jnp.int32, 16)
    one = jnp.ones((16,), jnp.int32)
    neg_inf = jnp.full((16,), -jnp.inf, jnp.float32)
    zero16 = jnp.zeros((16,), jnp.float32)

    obase = pl.multiple_of(sid * NODE_TILE, 8)
    pltpu.sync_copy(s_hbm.at[pl.ds(obase, NODE_TILE)], comb_v)
    pltpu.sync_copy(comb_v, s_sp.at[pl.ds(obase, NODE_TILE)])
    pltpu.sync_copy(t_hbm.at[pl.ds(obase, NODE_TILE)], comb_v)
    pltpu.sync_copy(comb_v, t_sp.at[pl.ds(obase, NODE_TILE)])
    pltpu.sync_copy(par_hbm, pv_v)

    def initm(i, _):
        acc_v[pl.ds(i * 16, 16)] = neg_inf
        return 0
    lax.fori_loop(0, NPAD // 16, initm, 0)
    plsc.subcore_barrier()

    a256 = pv_v[...]
    ebase = pl.multiple_of(sid * EA_TILE, 8)

    def seg_update(idx, val, bits0, is_max):
        def cond(b):
            return b > 0

        def step(b):
            act = ((b >> lanes) & 1) != 0
            cur = plsc.load_gather(acc_v, [idx], mask=act)
            plsc.store_scatter(acc_v, [idx], plsc.bitcast(lanes, jnp.float32),
                               mask=act)
            back = plsc.bitcast(plsc.load_gather(acc_v, [idx], mask=act),
                                jnp.int32)
            win = (back == lanes) & act
            newv = jnp.maximum(cur, val) if is_max else cur + val
            plsc.store_scatter(acc_v, [idx], newv, mask=win)
            rem = act & jnp.logical_not(win)
            return jnp.sum(jnp.where(rem, one << lanes, 0))

        lax.while_loop(cond, step, bits0)

    def gather128(table_sp, idx_ref, out_ref, s):
        descs = []
        for g in range(GSUB):
            descs.append(pltpu.async_copy(
                table_sp.at[idx_ref.at[pl.ds(g * 128, 128)]],
                out_ref.at[pl.ds(g * 128, 128)], s))
        return descs

    def load_edges(off):
        pltpu.sync_copy(src_hbm.at[pl.ds(off, CHUNK_A)], idx_s)
        pltpu.sync_copy(dst_hbm.at[pl.ds(off, CHUNK_A)], idx_d)
        pltpu.sync_copy(ea_hbm.at[pl.ds(off, CHUNK_A)], ea_v)
        ds1 = gather128(s_sp, idx_s, g1_v, sem)
        ds2 = gather128(t_sp, idx_d, g2_v, sem2)
        for d in ds1 + ds2:
            d.wait()

    def compute_e(j):
        sv = g1_v[pl.ds(j * 16, 16)]
        tv = g2_v[pl.ds(j * 16, 16)]
        eav = ea_v[pl.ds(j * 16, 16)]
        e = sv + tv + eav * a256
        return jnp.where(e >= 0, e, e * jnp.float32(0.2))

    def combine(is_max, dst_sp):
        for half in range(2):
            hbase = half * HNP
            pltpu.sync_copy(
                acc_v.at[pl.ds(hbase, HNP)],
                red_sp.at[pl.ds(pl.multiple_of(sid * HNP, 8), HNP)])
            plsc.subcore_barrier()

            @pl.when((sid // 8) == half)
            def _():
                lbase = pl.multiple_of((sid - half * 8) * NODE_TILE, 8)

                def cinit(i, _):
                    comb_v[pl.ds(i * 16, 16)] = (neg_inf if is_max
                                                 else zero16)
                    return 0
                lax.fori_loop(0, NODE_TILE // 16, cinit, 0)

                def creduce(t, _):
                    pltpu.sync_copy(
                        red_sp.at[pl.ds(
                            pl.multiple_of(t * HNP, 8) + lbase, NODE_TILE)],
                        tmp_v)

                    def vred(i, _):
                        a = comb_v[pl.ds(i * 16, 16)]
                        b = tmp_v[pl.ds(i * 16, 16)]
                        comb_v[pl.ds(i * 16, 16)] = (
                            jnp.maximum(a, b) if is_max else a + b)
                        return 0
                    lax.fori_loop(0, NODE_TILE // 16, vred, 0)
                    return 0
                lax.fori_loop(0, NSUB, creduce, 0)

                def cfin(i, _):
                    v = comb_v[pl.ds(i * 16, 16)]
                    if is_max:
                        v = jnp.where(v == neg_inf, zero16, v)
                    else:
                        v = jnp.float32(1.0) / jnp.maximum(
                            v, jnp.full((16,), 1e-12, jnp.float32))
                    comb_v[pl.ds(i * 16, 16)] = v
                    return 0
                lax.fori_loop(0, NODE_TILE // 16, cfin, 0)
                pltpu.sync_copy(comb_v, dst_sp.at[pl.ds(obase, NODE_TILE)])
            plsc.subcore_barrier()

    def chunk1(k, _):
        off = pl.multiple_of(ebase + k * CHUNK_A, 8)
        load_edges(off)

        def vloop(j, _):
            e = compute_e(j)
            idx = idx_d[pl.ds(j * 16, 16)]
            seg_update(idx, e, jnp.int32(0xFFFF), True)
            return 0

        lax.fori_loop(0, CHUNK_A // 16, vloop, 0)
        return 0

    lax.fori_loop(0, NCHUNK_A, chunk1, 0)

    combine(True, emax_sp)

    def initz(i, _):
        acc_v[pl.ds(i * 16, 16)] = zero16
        return 0
    lax.fori_loop(0, NPAD // 16, initz, 0)
    plsc.subcore_barrier()

    def chunk2(k, _):
        off = pl.multiple_of(ebase + k * CHUNK_A, 8)
        load_edges(off)
        for d in gather128(emax_sp, idx_d, g3_v, sem):
            d.wait()

        def vloop(j, _):
            e = compute_e(j)
            em = g3_v[pl.ds(j * 16, 16)]
            au = jnp.exp(e - em)
            idx = idx_d[pl.ds(j * 16, 16)]
            seg_update(idx, au, jnp.int32(0xFFFF), False)
            return 0

        lax.fori_loop(0, CHUNK_A // 16, vloop, 0)
        return 0

    lax.fori_loop(0, NCHUNK_A, chunk2, 0)

    combine(False, inv_sp)

    wbase = sid * EA_TILE + cid * EW_TILE

    def chunk3(k, _):
        off = pl.multiple_of(wbase + k * CHUNK_A, 8)
        load_edges(off)
        ds3 = gather128(emax_sp, idx_d, g3_v, sem)
        ds4 = gather128(inv_sp, idx_d, g4_v, sem2)
        for d in ds3 + ds4:
            d.wait()

        @plsc.parallel_loop(0, CHUNK_A // 16, unroll=4)
        def vloop(j):
            e = compute_e(j)
            em = g3_v[pl.ds(j * 16, 16)]
            iv = g4_v[pl.ds(j * 16, 16)]
            eav = ea_v[pl.ds(j * 16, 16)]
            valid = (off + j * 16 + lanes) < E
            alpha = jnp.where(valid, jnp.exp(e - em) * iv, zero16)
            ew = jnp.maximum(jnp.abs(eav), jnp.full((16,), 0.01, jnp.float32))
            g1_v[pl.ds(j * 16, 16)] = alpha
            g2_v[pl.ds(j * 16, 16)] = alpha * ew

        pltpu.sync_copy(g1_v, alpha_hbm.at[pl.ds(off, CHUNK_A)])
        pltpu.sync_copy(g2_v, coef_hbm.at[pl.ds(off, CHUNK_A)])
        return 0

    lax.fori_loop(0, NCHUNK_W, chunk3, 0)


def _sc_softmax(src, dst, ea, s, t, par):
    mesh = plsc.VectorSubcoreMesh(core_axis_name="c", subcore_axis_name="s")
    kern = functools.partial(
        pl.kernel,
        out_type=[
            jax.ShapeDtypeStruct((EPAD,), jnp.float32),
            jax.ShapeDtypeStruct((EPAD,), jnp.float32),
        ],
        mesh=mesh,
        compiler_params=pltpu.CompilerParams(needs_layout_passes=False),
        scratch_types=[
            pltpu.VMEM((CHUNK_A,), jnp.int32),
            pltpu.VMEM((CHUNK_A,), jnp.int32),
            pltpu.VMEM((CHUNK_A,), jnp.float32),
            pltpu.VMEM((CHUNK_A,), jnp.float32),
            pltpu.VMEM((CHUNK_A,), jnp.float32),
            pltpu.VMEM((CHUNK_A,), jnp.float32),
            pltpu.VMEM((CHUNK_A,), jnp.float32),
            pltpu.VMEM((NPAD,), jnp.float32),
            pltpu.VMEM((NODE_TILE,), jnp.float32),
            pltpu.VMEM((NODE_TILE,), jnp.float32),
            pltpu.VMEM((16,), jnp.float32),
            pltpu.VMEM_SHARED((NPAD,), jnp.float32),
            pltpu.VMEM_SHARED((NPAD,), jnp.float32),
            pltpu.VMEM_SHARED((NPAD,), jnp.float32),
            pltpu.VMEM_SHARED((NPAD,), jnp.float32),
            pltpu.VMEM_SHARED((NSUB * HNP,), jnp.float32),
            pltpu.SemaphoreType.DMA,
            pltpu.SemaphoreType.DMA,
        ],
    )(_softmax_body)
    return kern(src, dst, ea, s, t, par)


NPASS = 7
PASS_ROWS = NPAD // NPASS
EB_TILE = EPAD // (NSUB * NC)
CHUNK_B = 1568
NCHUNK_B = EB_TILE // CHUNK_B
STAGE_B = 1664
DRAIN_W = 56
TILE_ROWS = PASS_ROWS // NSUB


def _agg_body(src_hbm, dst_hbm, coef_hbm, h_hbm, opart_hbm,
              c_src, c_dst, c_cof, st_src, st_dst, st_cof,
              blki0, blkd0, blkc0, blki1, blkd1, blkc1,
              rows0_v, rows1_v, zero_v, acc_sp, semg0, semg1, sems0, sems1):
    cid = lax.axis_index("c")
    sid = lax.axis_index("s")
    lanes = lax.iota(jnp.int32, 16)
    zero16 = jnp.zeros((16,), jnp.float32)
    wid = cid * NSUB + sid
    tbase = pl.multiple_of(wid * EB_TILE, 8)
    rb0 = sid * TILE_ROWS

    def zinit(r, _):
        for j in range(8):
            zero_v[r, pl.ds(j * 16, 16)] = zero16
        return 0
    lax.fori_loop(0, DRAIN_W, zinit, 0)

    for p in range(NPASS):
        prow_base = p * PASS_ROWS
        for w in range(PASS_ROWS // NSUB // DRAIN_W):
            rs = pl.multiple_of(rb0 + w * DRAIN_W, 8)
            pltpu.sync_copy(zero_v, acc_sp.at[pl.ds(rs, DRAIN_W)])
        plsc.subcore_barrier()

        def chunkb(k, _):
            off = pl.multiple_of(tbase + k * CHUNK_B, 8)
            pltpu.sync_copy(src_hbm.at[pl.ds(off, CHUNK_B)], c_src)
            pltpu.sync_copy(dst_hbm.at[pl.ds(off, CHUNK_B)], c_dst)
            pltpu.sync_copy(coef_hbm.at[pl.ds(off, CHUNK_B)], c_cof)

            @plsc.parallel_loop(0, CHUNK_B // 16, unroll=4, carry=jnp.int32(0))
            def vstage(j, cnt):
                dl = c_dst[pl.ds(j * 16, 16)] - prow_base
                m = (dl >= 0) & (dl < PASS_ROWS)
                plsc.store_compressed(st_src.at[pl.ds(cnt, 16)],
                                      c_src[pl.ds(j * 16, 16)], mask=m)
                plsc.store_compressed(st_dst.at[pl.ds(cnt, 16)], dl, mask=m)
                plsc.store_compressed(st_cof.at[pl.ds(cnt, 16)],
                                      c_cof[pl.ds(j * 16, 16)], mask=m)
                return cnt + jnp.sum(m.astype(jnp.int32))

            cnt = vstage
            nblk = (cnt + 127) // 128

            def prep(b, blki, blkd, blkc, rows, semg, sems):
                @pl.when(b >= 2)
                def _():
                    pltpu.make_async_copy(rows, acc_sp.at[blkd], sems).wait()
                boff = b * 128
                for j in range(8):
                    pos = boff + j * 16 + lanes
                    vv = pos < cnt
                    sidx = st_src[pl.ds(boff + j * 16, 16)]
                    didx = st_dst[pl.ds(boff + j * 16, 16)]
                    cv = st_cof[pl.ds(boff + j * 16, 16)]
                    fb = j * 16 + lanes
                    blki[pl.ds(j * 16, 16)] = jnp.where(vv, sidx, fb)
                    blkd[pl.ds(j * 16, 16)] = jnp.where(
                        vv, didx, jnp.full((16,), PASS_ROWS, jnp.int32))
                    blkc[pl.ds(j * 16, 16)] = cv
                pltpu.async_copy(h_hbm.at[blki], rows, semg)

            def finish(b, blki, blkd, blkc, rows, semg, sems):
                pltpu.make_async_copy(h_hbm.at[blki], rows, semg).wait()
                nvalid = jnp.minimum(cnt - b * 128, 128)

                @plsc.parallel_loop(0, nvalid, unroll=4)
                def scale(r):
                    cvec = plsc.load_gather(
                        blkc, [jnp.full((16,), r, jnp.int32)])
                    for jj in range(8):
                        rows[r, pl.ds(jj * 16, 16)] = (
                            rows[r, pl.ds(jj * 16, 16)] * cvec)
                pltpu.async_copy(rows, acc_sp.at[blkd], sems, add=True)

            B0 = (blki0, blkd0, blkc0, rows0_v, semg0, sems0)
            B1 = (blki1, blkd1, blkc1, rows1_v, semg1, sems1)

            @pl.when(nblk > 0)
            def _():
                prep(0, *B0)

            def gpair(i, _):
                b1 = 2 * i + 1

                @pl.when(b1 < nblk)
                def _():
                    prep(b1, *B1)
                finish(b1 - 1, *B0)

                @pl.when(b1 < nblk)
                def _():
                    @pl.when(b1 + 1 < nblk)
                    def _():
                        prep(b1 + 1, *B0)
                    finish(b1, *B1)
                return 0

            lax.fori_loop(0, (nblk + 1) // 2, gpair, 0)

            @pl.when(nblk >= 2)
            def _():
                pltpu.make_async_copy(rows0_v, acc_sp.at[blkd0], sems0).wait()
                pltpu.make_async_copy(rows1_v, acc_sp.at[blkd1], sems1).wait()

            @pl.when(nblk == 1)
            def _():
                pltpu.make_async_copy(rows0_v, acc_sp.at[blkd0], sems0).wait()
            return 0

        lax.fori_loop(0, NCHUNK_B, chunkb, 0)
        plsc.subcore_barrier()

        for w in range(PASS_ROWS // NSUB // DRAIN_W):
            rs = pl.multiple_of(rb0 + w * DRAIN_W, 8)
            pltpu.sync_copy(acc_sp.at[pl.ds(rs, DRAIN_W)],
                            rows0_v.at[pl.ds(0, DRAIN_W)])
            pltpu.sync_copy(
                rows0_v.at[pl.ds(0, DRAIN_W)],
                opart_hbm.at[cid, pl.ds(pl.multiple_of(prow_base, 8) + rs,
                                        DRAIN_W)])
        plsc.subcore_barrier()


def _sc_aggregate(src, dst, coef, h):
    mesh = plsc.VectorSubcoreMesh(core_axis_name="c", subcore_axis_name="s")
    kern = functools.partial(
        pl.kernel,
        out_type=[jax.ShapeDtypeStruct((NC, NPAD, D), jnp.float32)],
        mesh=mesh,
        compiler_params=pltpu.CompilerParams(needs_layout_passes=False),
        scratch_types=[
            pltpu.VMEM((CHUNK_B,), jnp.int32),
            pltpu.VMEM((CHUNK_B,), jnp.int32),
            pltpu.VMEM((CHUNK_B,), jnp.float32),
            pltpu.VMEM((STAGE_B,), jnp.int32),
            pltpu.VMEM((STAGE_B,), jnp.int32),
            pltpu.VMEM((STAGE_B,), jnp.float32),
            pltpu.VMEM((128,), jnp.int32),
            pltpu.VMEM((128,), jnp.int32),
            pltpu.VMEM((128,), jnp.float32),
            pltpu.VMEM((128,), jnp.int32),
            pltpu.VMEM((128,), jnp.int32),
            pltpu.VMEM((128,), jnp.float32),
            pltpu.VMEM((128, D), jnp.float32),
            pltpu.VMEM((128, D), jnp.float32),
            pltpu.VMEM((DRAIN_W, D), jnp.float32),
            pltpu.VMEM_SHARED((PASS_ROWS + 8, D), jnp.float32),
            pltpu.SemaphoreType.DMA,
            pltpu.SemaphoreType.DMA,
            pltpu.SemaphoreType.DMA,
            pltpu.SemaphoreType.DMA,
        ],
    )(_agg_body)
    (opart,) = kern(src, dst, coef, h)
    return opart


def _combine_body(w_ref, o0_ref, o1_ref, b_ref, out_ref):
    o0 = o0_ref[0] + o0_ref[1]
    o1 = o1_ref[0] + o1_ref[1]
    out_ref[...] = w_ref[0] * o0 + w_ref[1] * o1 + b_ref[...]


def _combine(w, opart0, opart1, bias):
    grid = (N // N_NODES_BLK,)
    return pl.pallas_call(
        _combine_body,
        grid=grid,
        in_specs=[
            pl.BlockSpec(memory_space=pltpu.SMEM),
            pl.BlockSpec((NC, N_NODES_BLK, D), lambda i: (0, i, 0)),
            pl.BlockSpec((NC, N_NODES_BLK, D), lambda i: (0, i, 0)),
            pl.BlockSpec((1, D), lambda i: (0, 0)),
        ],
        out_specs=pl.BlockSpec((N_NODES_BLK, D), lambda i: (i, 0)),
        out_shape=jax.ShapeDtypeStruct((N, D), jnp.float32),
    )(w, opart0, opart1, bias.reshape(1, D))


def _attend_rel(h, s, t, edge_index, edge_attr, a_last):
    src = jnp.pad(edge_index[0], (0, EPAD - E))
    dst = jnp.pad(edge_index[1], (0, EPAD - E), constant_values=NPAD - 8)
    ea1 = jnp.pad(edge_attr[:, 0], (0, EPAD - E))
    sp = jnp.pad(s, (0, NPAD - N))
    tp = jnp.pad(t, (0, NPAD - N))
    par = jnp.full((16,), a_last, jnp.float32)
    alpha_p, coef_p = _sc_softmax(src, dst, ea1, sp, tp, par)
    opart = _sc_aggregate(src, dst, coef_p, h)
    return opart, alpha_p[:E]


def kernel(x, edge_index_r0, edge_attr_r0, edge_index_r1, edge_attr_r1,
           W0, W1, a0, a1, relation_logits, bias):
    h0, h1, st = _project(x, W0, W1, a0, a1)
    op0, alpha0 = _attend_rel(h0, st[:, 0], st[:, 1], edge_index_r0,
                              edge_attr_r0, a0[2 * D])
    op1, alpha1 = _attend_rel(h1, st[:, 2], st[:, 3], edge_index_r1,
                              edge_attr_r1, a1[2 * D])
    weights = jax.nn.softmax(relation_logits, axis=0)
    out = _combine(weights, op0, op1, bias)
    return (out, alpha0, alpha1)


# speedup vs baseline: 11.7983x; 1.0377x over previous
import functools
import jax
import jax.numpy as jnp
from jax import lax
from jax.experimental import pallas as pl
from jax.experimental.pallas import tpu as pltpu
from jax.experimental.pallas import tpu_sc as plsc

N = 50000
E = 400000
D = 128
NPAD = 50176
HNP = NPAD // 2
EPAD = 401408
NSUB = 16
NC = 2
NODE_TILE = NPAD // NSUB
EA_TILE = EPAD // NSUB
CHUNK_A = 1792
NCHUNK_A = EA_TILE // CHUNK_A
GSUB = CHUNK_A // 128
EW_TILE = EPAD // (NSUB * NC)
NCHUNK_W = EW_TILE // CHUNK_A

N_NODES_BLK = 2000


def _proj_body(x_ref, w0_ref, w1_ref, a2_ref, h0_ref, h1_ref, st_ref):
    x = x_ref[...]
    h0 = jnp.dot(x, w0_ref[...], preferred_element_type=jnp.float32)
    h1 = jnp.dot(x, w1_ref[...], preferred_element_type=jnp.float32)
    h0_ref[...] = h0
    h1_ref[...] = h1
    a2 = a2_ref[...]
    st0 = jnp.dot(h0, a2[:, 0:2], preferred_element_type=jnp.float32)
    st1 = jnp.dot(h1, a2[:, 2:4], preferred_element_type=jnp.float32)
    st_ref[...] = jnp.concatenate(
        [st0, st1, jnp.zeros_like(st0), jnp.zeros_like(st0)], axis=-1)


def _project(x, W0, W1, a0, a1):
    a2 = jnp.stack([a0[:D], a0[D:2 * D], a1[:D], a1[D:2 * D]], axis=-1)
    a2 = jnp.pad(a2, ((0, 0), (0, 4)))
    grid = (N // N_NODES_BLK,)
    h0, h1, st = pl.pallas_call(
        _proj_body,
        grid=grid,
        in_specs=[
            pl.BlockSpec((N_NODES_BLK, D), lambda i: (i, 0)),
            pl.BlockSpec((D, D), lambda i: (0, 0)),
            pl.BlockSpec((D, D), lambda i: (0, 0)),
            pl.BlockSpec((D, 8), lambda i: (0, 0)),
        ],
        out_specs=[
            pl.BlockSpec((N_NODES_BLK, D), lambda i: (i, 0)),
            pl.BlockSpec((N_NODES_BLK, D), lambda i: (i, 0)),
            pl.BlockSpec((N_NODES_BLK, 8), lambda i: (i, 0)),
        ],
        out_shape=[
            jax.ShapeDtypeStruct((N, D), jnp.float32),
            jax.ShapeDtypeStruct((N, D), jnp.float32),
            jax.ShapeDtypeStruct((N, 8), jnp.float32),
        ],
    )(x, W0, W1, a2)
    return h0, h1, st


def _softmax_body(src_hbm, dst_hbm, ea_hbm, s_hbm, t_hbm, par_hbm,
                  alpha_hbm, coef_hbm,
                  idx_s, idx_d, ea_v, g1_v, g2_v, g3_v, g4_v,
                  acc_v, comb_v, tmp_v, pv_v,
                  s_sp, t_sp, emax_sp, inv_sp, red_sp, sem, sem2, sem3):
    cid = lax.axis_index("c")
    sid = lax.axis_index("s")
    lanes = lax.iota(jnp.int32, 16)
    one = jnp.ones((16,), jnp.int32)
    neg_inf = jnp.full((16,), -jnp.inf, jnp.float32)
    zero16 = jnp.zeros((16,), jnp.float32)

    obase = pl.multiple_of(sid * NODE_TILE, 8)
    pltpu.sync_copy(s_hbm.at[pl.ds(obase, NODE_TILE)], comb_v)
    pltpu.sync_copy(comb_v, s_sp.at[pl.ds(obase, NODE_TILE)])
    pltpu.sync_copy(t_hbm.at[pl.ds(obase, NODE_TILE)], comb_v)
    pltpu.sync_copy(comb_v, t_sp.at[pl.ds(obase, NODE_TILE)])
    pltpu.sync_copy(par_hbm, pv_v)

    def initm(i, _):
        acc_v[pl.ds(i * 16, 16)] = neg_inf
        return 0
    lax.fori_loop(0, NPAD // 16, initm, 0)
    plsc.subcore_barrier()

    a256 = pv_v[...]
    ebase = pl.multiple_of(sid * EA_TILE, 8)

    def seg_update(idx, val, bits0, is_max):
        def cond(b):
            return b > 0

        def step(b):
            act = ((b >> lanes) & 1) != 0
            cur = plsc.load_gather(acc_v, [idx], mask=act)
            plsc.store_scatter(acc_v, [idx], plsc.bitcast(lanes, jnp.float32),
                               mask=act)
            back = plsc.bitcast(plsc.load_gather(acc_v, [idx], mask=act),
                                jnp.int32)
            win = (back == lanes) & act
            newv = jnp.maximum(cur, val) if is_max else cur + val
            plsc.store_scatter(acc_v, [idx], newv, mask=win)
            rem = act & jnp.logical_not(win)
            return jnp.sum(jnp.where(rem, one << lanes, 0))

        lax.while_loop(cond, step, bits0)

    def gather128(table_sp, idx_ref, out_ref, s):
        descs = []
        for g in range(GSUB):
            descs.append(pltpu.async_copy(
                table_sp.at[idx_ref.at[pl.ds(g * 128, 128)]],
                out_ref.at[pl.ds(g * 128, 128)], s))
        return descs

    def load_edges(off):
        cps = pltpu.async_copy(src_hbm.at[pl.ds(off, CHUNK_A)], idx_s, sem)
        cpd = pltpu.async_copy(dst_hbm.at[pl.ds(off, CHUNK_A)], idx_d, sem2)
        cpe = pltpu.async_copy(ea_hbm.at[pl.ds(off, CHUNK_A)], ea_v, sem3)
        cps.wait()
        cpd.wait()
        ds1 = gather128(s_sp, idx_s, g1_v, sem)
        ds2 = gather128(t_sp, idx_d, g2_v, sem2)
        cpe.wait()
        for d in ds1 + ds2:
            d.wait()

    def compute_e(j):
        sv = g1_v[pl.ds(j * 16, 16)]
        tv = g2_v[pl.ds(j * 16, 16)]
        eav = ea_v[pl.ds(j * 16, 16)]
        e = sv + tv + eav * a256
        return jnp.where(e >= 0, e, e * jnp.float32(0.2))

    def combine(is_max, dst_sp):
        for half in range(2):
            hbase = half * HNP
            pltpu.sync_copy(
                acc_v.at[pl.ds(hbase, HNP)],
                red_sp.at[pl.ds(pl.multiple_of(sid * HNP, 8), HNP)])
            plsc.subcore_barrier()

            @pl.when((sid // 8) == half)
            def _():
                lbase = pl.multiple_of((sid - half * 8) * NODE_TILE, 8)

                def cinit(i, _):
                    comb_v[pl.ds(i * 16, 16)] = (neg_inf if is_max
                                                 else zero16)
                    return 0
                lax.fori_loop(0, NODE_TILE // 16, cinit, 0)

                def creduce(t, _):
                    pltpu.sync_copy(
                        red_sp.at[pl.ds(
                            pl.multiple_of(t * HNP, 8) + lbase, NODE_TILE)],
                        tmp_v)

                    def vred(i, _):
                        a = comb_v[pl.ds(i * 16, 16)]
                        b = tmp_v[pl.ds(i * 16, 16)]
                        comb_v[pl.ds(i * 16, 16)] = (
                            jnp.maximum(a, b) if is_max else a + b)
                        return 0
                    lax.fori_loop(0, NODE_TILE // 16, vred, 0)
                    return 0
                lax.fori_loop(0, NSUB, creduce, 0)

                def cfin(i, _):
                    v = comb_v[pl.ds(i * 16, 16)]
                    if is_max:
                        v = jnp.where(v == neg_inf, zero16, v)
                    else:
                        v = jnp.float32(1.0) / jnp.maximum(
                            v, jnp.full((16,), 1e-12, jnp.float32))
                    comb_v[pl.ds(i * 16, 16)] = v
                    return 0
                lax.fori_loop(0, NODE_TILE // 16, cfin, 0)
                pltpu.sync_copy(comb_v, dst_sp.at[pl.ds(obase, NODE_TILE)])
            plsc.subcore_barrier()

    def chunk1(k, _):
        off = pl.multiple_of(ebase + k * CHUNK_A, 8)
        load_edges(off)

        def vloop(j, _):
            e = compute_e(j)
            idx = idx_d[pl.ds(j * 16, 16)]
            seg_update(idx, e, jnp.int32(0xFFFF), True)
            return 0

        lax.fori_loop(0, CHUNK_A // 16, vloop, 0)
        return 0

    lax.fori_loop(0, NCHUNK_A, chunk1, 0)

    combine(True, emax_sp)

    def initz(i, _):
        acc_v[pl.ds(i * 16, 16)] = zero16
        return 0
    lax.fori_loop(0, NPAD // 16, initz, 0)
    plsc.subcore_barrier()

    def chunk2(k, _):
        off = pl.multiple_of(ebase + k * CHUNK_A, 8)
        load_edges(off)
        for d in gather128(emax_sp, idx_d, g3_v, sem):
            d.wait()

        def vloop(j, _):
            e = compute_e(j)
            em = g3_v[pl.ds(j * 16, 16)]
            au = jnp.exp(e - em)
            idx = idx_d[pl.ds(j * 16, 16)]
            seg_update(idx, au, jnp.int32(0xFFFF), False)
            return 0

        lax.fori_loop(0, CHUNK_A // 16, vloop, 0)
        return 0

    lax.fori_loop(0, NCHUNK_A, chunk2, 0)

    combine(False, inv_sp)

    wbase = sid * EA_TILE + cid * EW_TILE

    def chunk3(k, _):
        off = pl.multiple_of(wbase + k * CHUNK_A, 8)
        load_edges(off)
        ds3 = gather128(emax_sp, idx_d, g3_v, sem)
        ds4 = gather128(inv_sp, idx_d, g4_v, sem2)
        for d in ds3 + ds4:
            d.wait()

        @plsc.parallel_loop(0, CHUNK_A // 16, unroll=4)
        def vloop(j):
            e = compute_e(j)
            em = g3_v[pl.ds(j * 16, 16)]
            iv = g4_v[pl.ds(j * 16, 16)]
            eav = ea_v[pl.ds(j * 16, 16)]
            valid = (off + j * 16 + lanes) < E
            alpha = jnp.where(valid, jnp.exp(e - em) * iv, zero16)
            ew = jnp.maximum(jnp.abs(eav), jnp.full((16,), 0.01, jnp.float32))
            g1_v[pl.ds(j * 16, 16)] = alpha
            g2_v[pl.ds(j * 16, 16)] = alpha * ew

        pltpu.sync_copy(g1_v, alpha_hbm.at[pl.ds(off, CHUNK_A)])
        pltpu.sync_copy(g2_v, coef_hbm.at[pl.ds(off, CHUNK_A)])
        return 0

    lax.fori_loop(0, NCHUNK_W, chunk3, 0)


def _sc_softmax(src, dst, ea, s, t, par):
    mesh = plsc.VectorSubcoreMesh(core_axis_name="c", subcore_axis_name="s")
    kern = functools.partial(
        pl.kernel,
        out_type=[
            jax.ShapeDtypeStruct((EPAD,), jnp.float32),
            jax.ShapeDtypeStruct((EPAD,), jnp.float32),
        ],
        mesh=mesh,
        compiler_params=pltpu.CompilerParams(needs_layout_passes=False),
        scratch_types=[
            pltpu.VMEM((CHUNK_A,), jnp.int32),
            pltpu.VMEM((CHUNK_A,), jnp.int32),
            pltpu.VMEM((CHUNK_A,), jnp.float32),
            pltpu.VMEM((CHUNK_A,), jnp.float32),
            pltpu.VMEM((CHUNK_A,), jnp.float32),
            pltpu.VMEM((CHUNK_A,), jnp.float32),
            pltpu.VMEM((CHUNK_A,), jnp.float32),
            pltpu.VMEM((NPAD,), jnp.float32),
            pltpu.VMEM((NODE_TILE,), jnp.float32),
            pltpu.VMEM((NODE_TILE,), jnp.float32),
            pltpu.VMEM((16,), jnp.float32),
            pltpu.VMEM_SHARED((NPAD,), jnp.float32),
            pltpu.VMEM_SHARED((NPAD,), jnp.float32),
            pltpu.VMEM_SHARED((NPAD,), jnp.float32),
            pltpu.VMEM_SHARED((NPAD,), jnp.float32),
            pltpu.VMEM_SHARED((NSUB * HNP,), jnp.float32),
            pltpu.SemaphoreType.DMA,
            pltpu.SemaphoreType.DMA,
            pltpu.SemaphoreType.DMA,
        ],
    )(_softmax_body)
    return kern(src, dst, ea, s, t, par)


NPASS = 7
PASS_ROWS = NPAD // NPASS
EB_TILE = EPAD // (NSUB * NC)
CHUNK_B = 1568
NCHUNK_B = EB_TILE // CHUNK_B
STAGE_B = 1664
DRAIN_W = 56
TILE_ROWS = PASS_ROWS // NSUB


def _agg_body(src_hbm, dst_hbm, coef_hbm, h_hbm, opart_hbm,
              c_src, c_dst, c_cof, st_src, st_dst, st_cof,
              blki0, blkd0, blkc0, blki1, blkd1, blkc1,
              rows0_v, rows1_v, zero_v, acc_sp, semg0, semg1, sems0, sems1):
    cid = lax.axis_index("c")
    sid = lax.axis_index("s")
    lanes = lax.iota(jnp.int32, 16)
    zero16 = jnp.zeros((16,), jnp.float32)
    wid = cid * NSUB + sid
    tbase = pl.multiple_of(wid * EB_TILE, 8)
    rb0 = sid * TILE_ROWS

    def zinit(r, _):
        for j in range(8):
            zero_v[r, pl.ds(j * 16, 16)] = zero16
        return 0
    lax.fori_loop(0, DRAIN_W, zinit, 0)

    for p in range(NPASS):
        prow_base = p * PASS_ROWS
        for w in range(PASS_ROWS // NSUB // DRAIN_W):
            rs = pl.multiple_of(rb0 + w * DRAIN_W, 8)
            pltpu.sync_copy(zero_v, acc_sp.at[pl.ds(rs, DRAIN_W)])
        plsc.subcore_barrier()

        def chunkb(k, _):
            off = pl.multiple_of(tbase + k * CHUNK_B, 8)
            pltpu.sync_copy(src_hbm.at[pl.ds(off, CHUNK_B)], c_src)
            pltpu.sync_copy(dst_hbm.at[pl.ds(off, CHUNK_B)], c_dst)
            pltpu.sync_copy(coef_hbm.at[pl.ds(off, CHUNK_B)], c_cof)

            @plsc.parallel_loop(0, CHUNK_B // 16, unroll=4, carry=jnp.int32(0))
            def vstage(j, cnt):
                dl = c_dst[pl.ds(j * 16, 16)] - prow_base
                m = (dl >= 0) & (dl < PASS_ROWS)
                plsc.store_compressed(st_src.at[pl.ds(cnt, 16)],
                                      c_src[pl.ds(j * 16, 16)], mask=m)
                plsc.store_compressed(st_dst.at[pl.ds(cnt, 16)], dl, mask=m)
                plsc.store_compressed(st_cof.at[pl.ds(cnt, 16)],
                                      c_cof[pl.ds(j * 16, 16)], mask=m)
                return cnt + jnp.sum(m.astype(jnp.int32))

            cnt = vstage
            nblk = (cnt + 127) // 128

            def prep(b, blki, blkd, blkc, rows, semg, sems):
                @pl.when(b >= 2)
                def _():
                    pltpu.make_async_copy(rows, acc_sp.at[blkd], sems).wait()
                boff = b * 128
                for j in range(8):
                    pos = boff + j * 16 + lanes
                    vv = pos < cnt
                    sidx = st_src[pl.ds(boff + j * 16, 16)]
                    didx = st_dst[pl.ds(boff + j * 16, 16)]
                    cv = st_cof[pl.ds(boff + j * 16, 16)]
                    fb = j * 16 + lanes
                    blki[pl.ds(j * 16, 16)] = jnp.where(vv, sidx, fb)
                    blkd[pl.ds(j * 16, 16)] = jnp.where(
                        vv, didx, jnp.full((16,), PASS_ROWS, jnp.int32))
                    blkc[pl.ds(j * 16, 16)] = cv
                pltpu.async_copy(h_hbm.at[blki], rows, semg)

            def finish(b, blki, blkd, blkc, rows, semg, sems):
                pltpu.make_async_copy(h_hbm.at[blki], rows, semg).wait()
                nvalid = jnp.minimum(cnt - b * 128, 128)

                @plsc.parallel_loop(0, nvalid, unroll=4)
                def scale(r):
                    cvec = plsc.load_gather(
                        blkc, [jnp.full((16,), r, jnp.int32)])
                    for jj in range(8):
                        rows[r, pl.ds(jj * 16, 16)] = (
                            rows[r, pl.ds(jj * 16, 16)] * cvec)
                pltpu.async_copy(rows, acc_sp.at[blkd], sems, add=True)

            B0 = (blki0, blkd0, blkc0, rows0_v, semg0, sems0)
            B1 = (blki1, blkd1, blkc1, rows1_v, semg1, sems1)

            @pl.when(nblk > 0)
            def _():
                prep(0, *B0)

            def gpair(i, _):
                b1 = 2 * i + 1

                @pl.when(b1 < nblk)
                def _():
                    prep(b1, *B1)
                finish(b1 - 1, *B0)

                @pl.when(b1 < nblk)
                def _():
                    @pl.when(b1 + 1 < nblk)
                    def _():
                        prep(b1 + 1, *B0)
                    finish(b1, *B1)
                return 0

            lax.fori_loop(0, (nblk + 1) // 2, gpair, 0)

            @pl.when(nblk >= 2)
            def _():
                pltpu.make_async_copy(rows0_v, acc_sp.at[blkd0], sems0).wait()
                pltpu.make_async_copy(rows1_v, acc_sp.at[blkd1], sems1).wait()

            @pl.when(nblk == 1)
            def _():
                pltpu.make_async_copy(rows0_v, acc_sp.at[blkd0], sems0).wait()
            return 0

        lax.fori_loop(0, NCHUNK_B, chunkb, 0)
        plsc.subcore_barrier()

        for w in range(PASS_ROWS // NSUB // DRAIN_W):
            rs = pl.multiple_of(rb0 + w * DRAIN_W, 8)
            pltpu.sync_copy(acc_sp.at[pl.ds(rs, DRAIN_W)],
                            rows0_v.at[pl.ds(0, DRAIN_W)])
            pltpu.sync_copy(
                rows0_v.at[pl.ds(0, DRAIN_W)],
                opart_hbm.at[cid, pl.ds(pl.multiple_of(prow_base, 8) + rs,
                                        DRAIN_W)])
        plsc.subcore_barrier()


def _sc_aggregate(src, dst, coef, h):
    mesh = plsc.VectorSubcoreMesh(core_axis_name="c", subcore_axis_name="s")
    kern = functools.partial(
        pl.kernel,
        out_type=[jax.ShapeDtypeStruct((NC, NPAD, D), jnp.float32)],
        mesh=mesh,
        compiler_params=pltpu.CompilerParams(needs_layout_passes=False),
        scratch_types=[
            pltpu.VMEM((CHUNK_B,), jnp.int32),
            pltpu.VMEM((CHUNK_B,), jnp.int32),
            pltpu.VMEM((CHUNK_B,), jnp.float32),
            pltpu.VMEM((STAGE_B,), jnp.int32),
            pltpu.VMEM((STAGE_B,), jnp.int32),
            pltpu.VMEM((STAGE_B,), jnp.float32),
            pltpu.VMEM((128,), jnp.int32),
            pltpu.VMEM((128,), jnp.int32),
            pltpu.VMEM((128,), jnp.float32),
            pltpu.VMEM((128,), jnp.int32),
            pltpu.VMEM((128,), jnp.int32),
            pltpu.VMEM((128,), jnp.float32),
            pltpu.VMEM((128, D), jnp.float32),
            pltpu.VMEM((128, D), jnp.float32),
            pltpu.VMEM((DRAIN_W, D), jnp.float32),
            pltpu.VMEM_SHARED((PASS_ROWS + 8, D), jnp.float32),
            pltpu.SemaphoreType.DMA,
            pltpu.SemaphoreType.DMA,
            pltpu.SemaphoreType.DMA,
            pltpu.SemaphoreType.DMA,
        ],
    )(_agg_body)
    (opart,) = kern(src, dst, coef, h)
    return opart


def _combine_body(w_ref, o0_ref, o1_ref, b_ref, out_ref):
    o0 = o0_ref[0] + o0_ref[1]
    o1 = o1_ref[0] + o1_ref[1]
    out_ref[...] = w_ref[0] * o0 + w_ref[1] * o1 + b_ref[...]


def _combine(w, opart0, opart1, bias):
    grid = (N // N_NODES_BLK,)
    return pl.pallas_call(
        _combine_body,
        grid=grid,
        in_specs=[
            pl.BlockSpec(memory_space=pltpu.SMEM),
            pl.BlockSpec((NC, N_NODES_BLK, D), lambda i: (0, i, 0)),
            pl.BlockSpec((NC, N_NODES_BLK, D), lambda i: (0, i, 0)),
            pl.BlockSpec((1, D), lambda i: (0, 0)),
        ],
        out_specs=pl.BlockSpec((N_NODES_BLK, D), lambda i: (i, 0)),
        out_shape=jax.ShapeDtypeStruct((N, D), jnp.float32),
    )(w, opart0, opart1, bias.reshape(1, D))


def _attend_rel(h, s, t, edge_index, edge_attr, a_last):
    src = jnp.pad(edge_index[0], (0, EPAD - E))
    dst = jnp.pad(edge_index[1], (0, EPAD - E), constant_values=NPAD - 8)
    ea1 = jnp.pad(edge_attr[:, 0], (0, EPAD - E))
    sp = jnp.pad(s, (0, NPAD - N))
    tp = jnp.pad(t, (0, NPAD - N))
    par = jnp.full((16,), a_last, jnp.float32)
    alpha_p, coef_p = _sc_softmax(src, dst, ea1, sp, tp, par)
    opart = _sc_aggregate(src, dst, coef_p, h)
    return opart, alpha_p[:E]


def kernel(x, edge_index_r0, edge_attr_r0, edge_index_r1, edge_attr_r1,
           W0, W1, a0, a1, relation_logits, bias):
    h0, h1, st = _project(x, W0, W1, a0, a1)
    op0, alpha0 = _attend_rel(h0, st[:, 0], st[:, 1], edge_index_r0,
                              edge_attr_r0, a0[2 * D])
    op1, alpha1 = _attend_rel(h1, st[:, 2], st[:, 3], edge_index_r1,
                              edge_attr_r1, a1[2 * D])
    weights = jax.nn.softmax(relation_logits, axis=0)
    out = _combine(weights, op0, op1, bias)
    return (out, alpha0, alpha1)
